# baseline (speedup 1.0000x reference)
import functools

import jax
import jax.numpy as jnp
from jax import lax
from jax.experimental import pallas as pl
from jax.experimental.pallas import tpu as pltpu
from jax.experimental.pallas import tpu_sc as plsc

N = 10000
E = 160000
G = 64
H = 128

EBLK = 640
NBLK = 400

NC = 2
NS = 16
NW = NC * NS
L = 16
CH = 128
CHG = 64
ZB = 624
ZREM = N - NS * ZB


def _mesh():
    return plsc.VectorSubcoreMesh(
        core_axis_name="c", subcore_axis_name="s", num_cores=NC, num_subcores=NS
    )


def _silu(x):
    return x * jax.nn.sigmoid(x)


def _mm(a, b):
    return jax.lax.dot_general(
        a, b, (((1,), (0,)), ((), ())), preferred_element_type=jnp.float32
    )


def _mmsel(oh, hi, lo):
    return _mm(oh, hi) + _mm(oh, lo)


def _bfsplit(x):
    hi = x.astype(jnp.bfloat16).astype(jnp.float32)
    return hi, x - hi


def _worker_range(w, nper, nchunk):
    base = nchunk // nper
    rem = nchunk % nper
    start = w * base + jnp.minimum(w, rem)
    n = base + (w < rem).astype(jnp.int32)
    return start, n, (base + 2) // 2


def _sc_gather(table, x4flat, elflat, eti, batchi, rowi, coli, ne):
    @functools.partial(
        pl.kernel,
        out_type=(
            jax.ShapeDtypeStruct((ne, H), jnp.float32),
            jax.ShapeDtypeStruct((ne, H), jnp.float32),
            jax.ShapeDtypeStruct((ne, H), jnp.float32),
        ),
        mesh=_mesh(),
        scratch_types=[
            pltpu.VMEM((CHG,), jnp.int32),
            pltpu.VMEM((CHG,), jnp.int32),
            pltpu.VMEM((CHG,), jnp.int32),
            pltpu.VMEM((CHG,), jnp.int32),
            pltpu.VMEM((CHG,), jnp.float32),
            pltpu.VMEM((CHG,), jnp.float32),
            pltpu.VMEM((CHG,), jnp.int32),
            pltpu.VMEM((CHG,), jnp.int32),
            pltpu.VMEM((CHG, H), jnp.float32),
            pltpu.VMEM((CHG, H), jnp.float32),
            pltpu.VMEM((CHG, H), jnp.float32),
            pltpu.VMEM((CHG, H), jnp.float32),
            pltpu.VMEM((CHG, H), jnp.float32),
            pltpu.VMEM((CHG, H), jnp.float32),
            pltpu.VMEM((N * 4,), jnp.float32),
            pltpu.VMEM((N,), jnp.int32),
            pltpu.SemaphoreType.DMA,
            pltpu.SemaphoreType.DMA,
            pltpu.SemaphoreType.DMA,
            pltpu.SemaphoreType.DMA,
        ],
        compiler_params=pltpu.CompilerParams(needs_layout_passes=False),
    )
    def k(table_h, x4_h, el_h, et_h, b_h, rowi_h, coli_h,
          outr_h, outc_h, aux_h,
          ir0, ir1, ic0, ic1, elb0, elb1, etb0, etb1,
          br0, br1, bc0, bc1, ax0, ax1, x4v, bv,
          sr0, sr1, sc0, sc1):
        irs = (ir0, ir1)
        ics = (ic0, ic1)
        elbs = (elb0, elb1)
        etbs = (etb0, etb1)
        brs = (br0, br1)
        bcs = (bc0, bc1)
        axs = (ax0, ax1)
        srs = (sr0, sr1)
        scs = (sc0, sc1)
        c = lax.axis_index("c")
        s = lax.axis_index("s")
        w = s * NC + c
        start, n, trips = _worker_range(w, NW, ne // CHG)
        pltpu.sync_copy(x4_h, x4v)
        pltpu.sync_copy(b_h, bv)
        iota = lax.iota(jnp.int32, L)

        def load_and_fire(kk, b):
            base = (start + kk) * CHG
            pltpu.sync_copy(rowi_h.at[pl.ds(base, CHG)], irs[b])
            pltpu.sync_copy(coli_h.at[pl.ds(base, CHG)], ics[b])
            pltpu.sync_copy(el_h.at[pl.ds(base, CHG)], elbs[b])
            pltpu.sync_copy(et_h.at[pl.ds(base, CHG)], etbs[b])
            pltpu.async_copy(table_h.at[irs[b]], brs[b], srs[b])
            pltpu.async_copy(table_h.at[ics[b]], bcs[b], scs[b])

        for b in range(2):
            @pl.when(b < n)
            def _():
                load_and_fire(b, b)

        def slot(i2, b):
            kk = i2 * 2 + b

            @pl.when(kk < n)
            def _():
                base = (start + kk) * CHG
                for j in range(CHG // L):
                    sl = pl.ds(j * L, L)
                    rv = irs[b][sl]
                    cv = ics[b][sl]
                    rv4 = rv * 4
                    cv4 = cv * 4
                    erow = iota + j * L
                    rel = []
                    for nn in range(3):
                        rn = (plsc.load_gather(x4v, [rv4 + nn])
                              - plsc.load_gather(x4v, [cv4 + nn]))
                        rel.append(rn)
                        plsc.store_scatter(axs[b], [erow, jnp.full((L,), nn, jnp.int32)], rn)
                    d2 = rel[0] * rel[0] + rel[1] * rel[1] + rel[2] * rel[2]
                    plsc.store_scatter(axs[b], [erow, jnp.full((L,), 3, jnp.int32)], d2)
                    e2g = plsc.load_gather(bv, [rv]).astype(jnp.float32)
                    plsc.store_scatter(axs[b], [erow, jnp.full((L,), 4, jnp.int32)], e2g)
                    plsc.store_scatter(axs[b], [erow, jnp.full((L,), 5, jnp.int32)], elbs[b][sl])
                    plsc.store_scatter(axs[b], [erow, jnp.full((L,), 6, jnp.int32)],
                                       etbs[b][sl].astype(jnp.float32))
                pltpu.make_async_copy(table_h.at[irs[b]], brs[b],
                                      srs[b]).wait()
                pltpu.make_async_copy(table_h.at[ics[b]], bcs[b],
                                      scs[b]).wait()
                pltpu.sync_copy(brs[b], outr_h.at[pl.ds(base, CHG)])
                pltpu.sync_copy(bcs[b], outc_h.at[pl.ds(base, CHG)])
                pltpu.sync_copy(axs[b], aux_h.at[pl.ds(base, CHG)])

                @pl.when(kk + 2 < n)
                def _():
                    load_and_fire(kk + 2, b)

        def body(i2, carry):
            slot(i2, 0)
            slot(i2, 1)
            return carry

        lax.fori_loop(0, trips, body, 0)

    return k(table, x4flat, elflat, eti, batchi, rowi, coli)


def _sc_scatter(m, rc128, rowi, zrows, ne):
    @functools.partial(
        pl.kernel,
        out_type=jax.ShapeDtypeStruct((NC, N, H), jnp.float32),
        mesh=_mesh(),
        scratch_types=[
            pltpu.VMEM_SHARED((N, H), jnp.float32),
            pltpu.VMEM((CH, H), jnp.float32),
            pltpu.VMEM((CH, H), jnp.float32),
            pltpu.VMEM((CH,), jnp.int32),
            pltpu.VMEM((CH,), jnp.int32),
            pltpu.SemaphoreType.DMA,
            pltpu.SemaphoreType.DMA,
        ],
    )
    def k(m_h, rc_h, rowi_h, z_h, out_h, acc, buf0, buf1, iv0, iv1, sa0, sa1):
        bufs = (buf0, buf1)
        ivs = (iv0, iv1)
        sas = (sa0, sa1)
        c = lax.axis_index("c")
        s = lax.axis_index("s")
        pltpu.sync_copy(z_h, acc.at[pl.ds(s * ZB, ZB)])

        @pl.when(s == 0)
        def _():
            pltpu.sync_copy(z_h.at[pl.ds(0, ZREM)], acc.at[pl.ds(NS * ZB, ZREM)])

        plsc.subcore_barrier()
        start, n, trips = _worker_range(s, NS, ne // CH)

        def wait_add(b):
            pltpu.make_async_copy(bufs[b], acc.at[ivs[b]], sas[b]).wait()

        def slot(i2, b, fill):
            kk = i2 * 2 + b

            @pl.when(kk < n)
            def _():
                @pl.when(kk >= 2)
                def _():
                    wait_add(b)

                base = (start + kk) * CH
                pltpu.sync_copy(rowi_h.at[pl.ds(base, CH)], ivs[b])
                fill(base, b)
                pltpu.async_copy(bufs[b], acc.at[ivs[b]], sas[b], add=True)

        def run(fill):
            def body(i2, carry):
                slot(i2, 0, fill)
                slot(i2, 1, fill)
                return carry

            lax.fori_loop(0, trips, body, 0)
            for b in range(2):
                @pl.when(n > b)
                def _():
                    wait_add(b)

        @pl.when(c == 0)
        def _():
            def fill_m(base, b):
                pltpu.sync_copy(m_h.at[pl.ds(base, CH)], bufs[b])

            run(fill_m)

        @pl.when(c == 1)
        def _():
            def fill_rc(base, b):
                pltpu.sync_copy(rc_h.at[pl.ds(base, CH)], bufs[b])

            run(fill_rc)

        plsc.subcore_barrier()
        pltpu.sync_copy(acc.at[pl.ds(s * ZB, ZB)],
                        out_h.at[c, pl.ds(s * ZB, ZB)])

        @pl.when(s == 0)
        def _():
            pltpu.sync_copy(acc.at[pl.ds(NS * ZB, ZREM)],
                            out_h.at[c, pl.ds(NS * ZB, ZREM)])

    return k(m, rc128, rowi, zrows)


def _tc_prelude(time_step2, p):
    def body(ts_r, wt0_r, bt0_r, wt1_r, bt1_r, wtp_r, btp_r, t_out):
        half = H // 2
        i64f = lax.broadcasted_iota(jnp.int32, (1, half), 1).astype(jnp.float32)
        freqs = jnp.exp(-jnp.log(10000.0) * i64f / (half - 1))
        args = ts_r[...].astype(jnp.float32) * freqs
        temb = jnp.concatenate([jnp.sin(args), jnp.cos(args)], axis=1)
        t = jax.nn.relu(_mm(temb, wt0_r[...]) + bt0_r[...])
        t = jax.nn.relu(_mm(t, wt1_r[...]) + bt1_r[...])
        t_out[...] = _mm(t, wtp_r[...]) + btp_r[...]

    return pl.pallas_call(
        body,
        out_shape=jax.ShapeDtypeStruct((G, H), jnp.float32),
    )(time_step2, p["Wt0"], p["bt0"].reshape(1, -1), p["Wt1"],
      p["bt1"].reshape(1, -1), p["Wtp"], p["btp"].reshape(1, -1))


def _tc_node_prep(atom_type, batch2, t, p):
    def body(at_r, b_r, th_r, tl_r, wa_r, wb_r, bin_r, out_r):
        gi = lax.broadcasted_iota(jnp.int32, (1, G), 1)
        oh = (b_r[...] == gi).astype(jnp.float32)
        tn = _mmsel(oh, th_r[...], tl_r[...])
        out_r[...] = _mm(at_r[...], wa_r[...]) + _mm(tn, wb_r[...]) + bin_r[...]

    grid = (N // NBLK,)
    return pl.pallas_call(
        body,
        grid=grid,
        in_specs=[
            pl.BlockSpec((NBLK, H), lambda i: (i, 0)),
            pl.BlockSpec((NBLK, 1), lambda i: (i, 0)),
            pl.BlockSpec((G, H), lambda i: (0, 0)),
            pl.BlockSpec((G, H), lambda i: (0, 0)),
            pl.BlockSpec((H, H), lambda i: (0, 0)),
            pl.BlockSpec((H, H), lambda i: (0, 0)),
            pl.BlockSpec((1, H), lambda i: (0, 0)),
        ],
        out_specs=pl.BlockSpec((NBLK, H), lambda i: (i, 0)),
        out_shape=jax.ShapeDtypeStruct((N, H), jnp.float32),
    )(atom_type, batch2, *_bfsplit(t), p["Win"][:H], p["Win"][H:],
      p["bin"].reshape(1, -1))


def _ea_from_aux(aux, th_r, tl_r, we1_r, be1_r, we2_r, be2_r, eh_r, el_r):
    el = aux[:, 5:6]
    etv = aux[:, 6:7]
    e2g = aux[:, 4:5]
    r1 = jax.nn.relu(el * we1_r[...] + be1_r[...])
    ea0 = _mm(r1, we2_r[...]) + be2_r[...]
    ei = lax.broadcasted_iota(jnp.int32, (1, 8), 1).astype(jnp.float32)
    ohe = (etv == ei).astype(jnp.float32)
    gi = lax.broadcasted_iota(jnp.int32, (1, G), 1).astype(jnp.float32)
    ohg = (e2g == gi).astype(jnp.float32)
    return (ea0 * _mmsel(ohe, eh_r[...], el_r[...])
            + _mmsel(ohg, th_r[...], tl_r[...]))


_EAW = None


def _ea_specs():
    W = pl.BlockSpec((H, H), lambda i: (0, 0))
    b1 = pl.BlockSpec((1, H), lambda i: (0, 0))
    return [
        pl.BlockSpec((G, H), lambda i: (0, 0)),
        pl.BlockSpec((G, H), lambda i: (0, 0)),
        pl.BlockSpec((1, H), lambda i: (0, 0)),
        b1, W, b1,
        pl.BlockSpec((8, H), lambda i: (0, 0)),
        pl.BlockSpec((8, H), lambda i: (0, 0)),
    ]


def _ea_args(t, p):
    th, tl = _bfsplit(t)
    eh, el_ = _bfsplit(p["emb_et"])
    return (th, tl, p["We1"], p["be1"].reshape(1, -1), p["We2"],
            p["be2"].reshape(1, -1), eh, el_)


def _tc_edge_mlp(hr, hc, aux, t, p, lp):
    def body(hr_r, hc_r, aux_r, th_r, tl_r, we1_r, be1_r, we2_r, be2_r,
             eh_r, el_r,
             w1r_r, w1c_r, w1d_r, w1e_r, bm1_r, wm2_r, bm2_r,
             wg_r, bg_r, wc1_r, wc2_r, m_out, rc_out):
        aux = aux_r[...]
        rel = aux[:, 0:3]
        d2 = aux[:, 3:4]
        ea = _ea_from_aux(aux, th_r, tl_r, we1_r, be1_r, we2_r, be2_r,
                          eh_r, el_r)
        d2b = d2.astype(jnp.bfloat16).astype(jnp.float32)
        w1db = w1d_r[...].astype(jnp.bfloat16).astype(jnp.float32)
        m = (_mm(hr_r[...], w1r_r[...]) + _mm(hc_r[...], w1c_r[...])
             + d2b * w1db + _mm(ea, w1e_r[...]) + bm1_r[...])
        m = _silu(m)
        m = _silu(_mm(m, wm2_r[...]) + bm2_r[...])
        m = m * jax.nn.sigmoid(_mm(m, wg_r[...]) + bg_r[...])
        coef = _mm(_silu(_mm(m, wc1_r[...])), wc2_r[...])
        reln = rel / (jnp.sqrt(d2) + 1.0)
        m_out[...] = m
        rc_out[...] = jnp.concatenate(
            [reln * coef, jnp.zeros((EBLK, H - 3), jnp.float32)], axis=1)

    ne = hr.shape[0]
    W = pl.BlockSpec((H, H), lambda i: (0, 0))
    b1 = pl.BlockSpec((1, H), lambda i: (0, 0))
    grid = (ne // EBLK,)
    return pl.pallas_call(
        body,
        grid=grid,
        in_specs=[
            pl.BlockSpec((EBLK, H), lambda i: (i, 0)),
            pl.BlockSpec((EBLK, H), lambda i: (i, 0)),
            pl.BlockSpec((EBLK, H), lambda i: (i, 0)),
        ] + _ea_specs() + [
            W, W, b1, W, b1, W, b1,
            pl.BlockSpec((H, 1), lambda i: (0, 0)),
            pl.BlockSpec((1, 1), lambda i: (0, 0)),
            W,
            pl.BlockSpec((H, 1), lambda i: (0, 0)),
        ],
        out_specs=(
            pl.BlockSpec((EBLK, H), lambda i: (i, 0)),
            pl.BlockSpec((EBLK, H), lambda i: (i, 0)),
        ),
        out_shape=(
            jax.ShapeDtypeStruct((ne, H), jnp.float32),
            jax.ShapeDtypeStruct((ne, H), jnp.float32),
        ),
    )(hr, hc, aux, *_ea_args(t, p),
      lp["Wm1"][:H], lp["Wm1"][H:2 * H], lp["Wm1"][2 * H:2 * H + 1],
      lp["Wm1"][2 * H + 1:], lp["bm1"].reshape(1, -1),
      lp["Wm2"], lp["bm2"].reshape(1, -1),
      lp["Wg"], lp["bg"].reshape(1, 1),
      lp["Wc1"], lp["Wc2"])


def _tc_node_update(table, x4, pA, pB, lp):
    def body(tbl_r, x4_r, pa_r, pb_r, whh_r, wha_r, bh1_r, wh2_r, bh2_r,
             h_out, x_out):
        pa = pa_r[...]
        pb = pb_r[...]
        agg = pa[0] + pb[0]
        h = tbl_r[...]
        upd = _mm(_silu(_mm(h, whh_r[...]) + _mm(agg, wha_r[...]) + bh1_r[...]),
                  wh2_r[...]) + bh2_r[...]
        h_out[...] = h + upd
        x_out[...] = x4_r[...] + (pa[1] + pb[1])[:, :4]

    W = pl.BlockSpec((H, H), lambda i: (0, 0))
    b1 = pl.BlockSpec((1, H), lambda i: (0, 0))
    grid = (N // NBLK,)
    return pl.pallas_call(
        body,
        grid=grid,
        in_specs=[
            pl.BlockSpec((NBLK, H), lambda i: (i, 0)),
            pl.BlockSpec((NBLK, 4), lambda i: (i, 0)),
            pl.BlockSpec((NC, NBLK, H), lambda i: (0, i, 0)),
            pl.BlockSpec((NC, NBLK, H), lambda i: (0, i, 0)),
            W, W, b1, W, b1,
        ],
        out_specs=(
            pl.BlockSpec((NBLK, H), lambda i: (i, 0)),
            pl.BlockSpec((NBLK, 4), lambda i: (i, 0)),
        ),
        out_shape=(
            jax.ShapeDtypeStruct((N, H), jnp.float32),
            jax.ShapeDtypeStruct((N, 4), jnp.float32),
        ),
    )(table, x4, pA, pB, lp["Wh1"][:H], lp["Wh1"][H:],
      lp["bh1"].reshape(1, -1), lp["Wh2"], lp["bh2"].reshape(1, -1))


def _tc_final(hr, hc, aux, t, p):
    def body(hr_r, hc_r, aux_r, th_r, tl_r, we1_r, be1_r, we2_r, be2_r,
             eh_r, el_r,
             wd1h_r, wd1e_r, bd1_r, wd2_r, bd2_r, wd3t_r, bd3_r, out_r):
        ea = _ea_from_aux(aux_r[...], th_r, tl_r, we1_r, be1_r, we2_r,
                          be2_r, eh_r, el_r)
        g1 = jax.nn.relu(_mm(hr_r[...] * hc_r[...], wd1h_r[...])
                         + _mm(ea, wd1e_r[...]) + bd1_r[...])
        g2 = jax.nn.relu(_mm(g1, wd2_r[...]) + bd2_r[...])
        out_r[0] = _mm(wd3t_r[...], g2.T) + bd3_r[...]

    ne = hr.shape[0]
    grid = (ne // EBLK,)
    return pl.pallas_call(
        body,
        grid=grid,
        in_specs=[
            pl.BlockSpec((EBLK, H), lambda i: (i, 0)),
            pl.BlockSpec((EBLK, H), lambda i: (i, 0)),
            pl.BlockSpec((EBLK, H), lambda i: (i, 0)),
        ] + _ea_specs() + [
            pl.BlockSpec((H, H), lambda i: (0, 0)),
            pl.BlockSpec((H, H), lambda i: (0, 0)),
            pl.BlockSpec((1, H), lambda i: (0, 0)),
            pl.BlockSpec((H, H // 2), lambda i: (0, 0)),
            pl.BlockSpec((1, H // 2), lambda i: (0, 0)),
            pl.BlockSpec((1, H // 2), lambda i: (0, 0)),
            pl.BlockSpec((1, 1), lambda i: (0, 0)),
        ],
        out_specs=pl.BlockSpec((1, 1, EBLK), lambda i: (i, 0, 0)),
        out_shape=jax.ShapeDtypeStruct((ne // EBLK, 1, EBLK), jnp.float32),
    )(hr, hc, aux, *_ea_args(t, p),
      p["Wd1"][:H], p["Wd1"][H:], p["bd1"].reshape(1, -1),
      p["Wd2"], p["bd2"].reshape(1, -1), p["Wd3"].reshape(1, -1),
      p["bd3"].reshape(1, 1))


def kernel(atom_type, pos, bond_index, bond_type, batch, time_step, edge_type,
           edge_index, edge_length, params):
    p = params
    row = edge_index[0].astype(jnp.int32)
    col = edge_index[1].astype(jnp.int32)
    batchi = batch.astype(jnp.int32)
    batch2 = batchi.reshape(N, 1)
    ts2 = time_step.astype(jnp.int32).reshape(G, 1)
    eti = edge_type.astype(jnp.int32)
    elflat = edge_length.reshape(E)
    x4 = jnp.pad(pos, ((0, 0), (0, 1)))
    zrows = jnp.zeros((ZB, H), jnp.float32)

    t = _tc_prelude(ts2, p)
    table = _tc_node_prep(atom_type, batch2, t, p)

    E2 = E // 2
    halves = []
    for h0 in (0, E2):
        halves.append(dict(
            row=row[h0:h0 + E2], col=col[h0:h0 + E2],
            el=elflat[h0:h0 + E2], et=eti[h0:h0 + E2]))

    def gath(hv):
        return _sc_gather(table, x4.reshape(-1), hv["el"], hv["et"], batchi,
                          hv["row"], hv["col"], E2)

    for lp in p["layers"]:
        gA = gath(halves[0])
        gB = gath(halves[1])
        mA, rcA = _tc_edge_mlp(*gA, t, p, lp)
        sA = _sc_scatter(mA, rcA, halves[0]["row"], zrows, E2)
        mB, rcB = _tc_edge_mlp(*gB, t, p, lp)
        sB = _sc_scatter(mB, rcB, halves[1]["row"], zrows, E2)
        table, x4 = _tc_node_update(table, x4, sA, sB, lp)

    gA = gath(halves[0])
    gB = gath(halves[1])
    oA = _tc_final(*gA, t, p)
    oB = _tc_final(*gB, t, p)
    edge_inv = jnp.concatenate([oA, oB], axis=0).reshape(E, 1)

    local_edge_mask = edge_type == 0
    return (edge_inv, edge_index, edge_type, edge_length, local_edge_mask)


# speedup vs baseline: 1.0008x; 1.0008x over previous
import functools

import jax
import jax.numpy as jnp
from jax import lax
from jax.experimental import pallas as pl
from jax.experimental.pallas import tpu as pltpu
from jax.experimental.pallas import tpu_sc as plsc

N = 10000
E = 160000
G = 64
H = 128

EBLK = 640
NBLK = 400

NC = 2
NS = 16
NW = NC * NS
L = 16
CH = 128
CHG = 64
ZB = 624
ZREM = N - NS * ZB


def _mesh():
    return plsc.VectorSubcoreMesh(
        core_axis_name="c", subcore_axis_name="s", num_cores=NC, num_subcores=NS
    )


def _silu(x):
    return x * jax.nn.sigmoid(x)


def _mm(a, b):
    return jax.lax.dot_general(
        a, b, (((1,), (0,)), ((), ())), preferred_element_type=jnp.float32
    )


def _mmsel(oh, table):
    hi = table.astype(jnp.bfloat16).astype(jnp.float32)
    lo = table - hi
    return _mm(oh, hi) + _mm(oh, lo)


def _worker_range(w, nper, nchunk):
    base = nchunk // nper
    rem = nchunk % nper
    start = w * base + jnp.minimum(w, rem)
    n = base + (w < rem).astype(jnp.int32)
    return start, n, (base + 2) // 2


def _sc_gather(table, x4flat, elflat, eti, batchi, rowi, coli, ne):
    @functools.partial(
        pl.kernel,
        out_type=(
            jax.ShapeDtypeStruct((ne, H), jnp.float32),
            jax.ShapeDtypeStruct((ne, H), jnp.float32),
            jax.ShapeDtypeStruct((ne, H), jnp.float32),
        ),
        mesh=_mesh(),
        scratch_types=[
            pltpu.VMEM((CHG,), jnp.int32),
            pltpu.VMEM((CHG,), jnp.int32),
            pltpu.VMEM((CHG,), jnp.int32),
            pltpu.VMEM((CHG,), jnp.int32),
            pltpu.VMEM((CHG,), jnp.float32),
            pltpu.VMEM((CHG,), jnp.float32),
            pltpu.VMEM((CHG,), jnp.int32),
            pltpu.VMEM((CHG,), jnp.int32),
            pltpu.VMEM((CHG, H), jnp.float32),
            pltpu.VMEM((CHG, H), jnp.float32),
            pltpu.VMEM((CHG, H), jnp.float32),
            pltpu.VMEM((CHG, H), jnp.float32),
            pltpu.VMEM((CHG, H), jnp.float32),
            pltpu.VMEM((CHG, H), jnp.float32),
            pltpu.VMEM((N * 4,), jnp.float32),
            pltpu.VMEM((N,), jnp.int32),
            pltpu.SemaphoreType.DMA,
            pltpu.SemaphoreType.DMA,
            pltpu.SemaphoreType.DMA,
            pltpu.SemaphoreType.DMA,
        ],
        compiler_params=pltpu.CompilerParams(needs_layout_passes=False),
    )
    def k(table_h, x4_h, el_h, et_h, b_h, rowi_h, coli_h,
          outr_h, outc_h, aux_h,
          ir0, ir1, ic0, ic1, elb0, elb1, etb0, etb1,
          br0, br1, bc0, bc1, ax0, ax1, x4v, bv,
          sr0, sr1, sc0, sc1):
        irs = (ir0, ir1)
        ics = (ic0, ic1)
        elbs = (elb0, elb1)
        etbs = (etb0, etb1)
        brs = (br0, br1)
        bcs = (bc0, bc1)
        axs = (ax0, ax1)
        srs = (sr0, sr1)
        scs = (sc0, sc1)
        c = lax.axis_index("c")
        s = lax.axis_index("s")
        w = s * NC + c
        start, n, trips = _worker_range(w, NW, ne // CHG)
        pltpu.sync_copy(x4_h, x4v)
        pltpu.sync_copy(b_h, bv)
        iota = lax.iota(jnp.int32, L)

        def load_and_fire(kk, b):
            base = (start + kk) * CHG
            pltpu.sync_copy(rowi_h.at[pl.ds(base, CHG)], irs[b])
            pltpu.sync_copy(coli_h.at[pl.ds(base, CHG)], ics[b])
            pltpu.sync_copy(el_h.at[pl.ds(base, CHG)], elbs[b])
            pltpu.sync_copy(et_h.at[pl.ds(base, CHG)], etbs[b])
            pltpu.async_copy(table_h.at[irs[b]], brs[b], srs[b])
            pltpu.async_copy(table_h.at[ics[b]], bcs[b], scs[b])

        for b in range(2):
            @pl.when(b < n)
            def _():
                load_and_fire(b, b)

        def slot(i2, b):
            kk = i2 * 2 + b

            @pl.when(kk < n)
            def _():
                base = (start + kk) * CHG
                for j in range(CHG // L):
                    sl = pl.ds(j * L, L)
                    rv = irs[b][sl]
                    cv = ics[b][sl]
                    rv4 = rv * 4
                    cv4 = cv * 4
                    erow = iota + j * L
                    rel = []
                    for nn in range(3):
                        rn = (plsc.load_gather(x4v, [rv4 + nn])
                              - plsc.load_gather(x4v, [cv4 + nn]))
                        rel.append(rn)
                        plsc.store_scatter(axs[b], [erow, jnp.full((L,), nn, jnp.int32)], rn)
                    d2 = rel[0] * rel[0] + rel[1] * rel[1] + rel[2] * rel[2]
                    plsc.store_scatter(axs[b], [erow, jnp.full((L,), 3, jnp.int32)], d2)
                    e2g = plsc.load_gather(bv, [rv]).astype(jnp.float32)
                    plsc.store_scatter(axs[b], [erow, jnp.full((L,), 4, jnp.int32)], e2g)
                    plsc.store_scatter(axs[b], [erow, jnp.full((L,), 5, jnp.int32)], elbs[b][sl])
                    plsc.store_scatter(axs[b], [erow, jnp.full((L,), 6, jnp.int32)],
                                       etbs[b][sl].astype(jnp.float32))
                pltpu.make_async_copy(table_h.at[irs[b]], brs[b],
                                      srs[b]).wait()
                pltpu.make_async_copy(table_h.at[ics[b]], bcs[b],
                                      scs[b]).wait()
                pltpu.sync_copy(brs[b], outr_h.at[pl.ds(base, CHG)])
                pltpu.sync_copy(bcs[b], outc_h.at[pl.ds(base, CHG)])
                pltpu.sync_copy(axs[b], aux_h.at[pl.ds(base, CHG)])

                @pl.when(kk + 2 < n)
                def _():
                    load_and_fire(kk + 2, b)

        def body(i2, carry):
            slot(i2, 0)
            slot(i2, 1)
            return carry

        lax.fori_loop(0, trips, body, 0)

    return k(table, x4flat, elflat, eti, batchi, rowi, coli)


def _sc_scatter(m, rc128, rowi, zrows, ne):
    @functools.partial(
        pl.kernel,
        out_type=jax.ShapeDtypeStruct((NC, N, H), jnp.float32),
        mesh=_mesh(),
        scratch_types=[
            pltpu.VMEM_SHARED((N, H), jnp.float32),
            pltpu.VMEM((CH, H), jnp.float32),
            pltpu.VMEM((CH, H), jnp.float32),
            pltpu.VMEM((CH,), jnp.int32),
            pltpu.VMEM((CH,), jnp.int32),
            pltpu.SemaphoreType.DMA,
            pltpu.SemaphoreType.DMA,
        ],
    )
    def k(m_h, rc_h, rowi_h, z_h, out_h, acc, buf0, buf1, iv0, iv1, sa0, sa1):
        bufs = (buf0, buf1)
        ivs = (iv0, iv1)
        sas = (sa0, sa1)
        c = lax.axis_index("c")
        s = lax.axis_index("s")
        pltpu.sync_copy(z_h, acc.at[pl.ds(s * ZB, ZB)])

        @pl.when(s == 0)
        def _():
            pltpu.sync_copy(z_h.at[pl.ds(0, ZREM)], acc.at[pl.ds(NS * ZB, ZREM)])

        plsc.subcore_barrier()
        start, n, trips = _worker_range(s, NS, ne // CH)

        def wait_add(b):
            pltpu.make_async_copy(bufs[b], acc.at[ivs[b]], sas[b]).wait()

        def slot(i2, b, fill):
            kk = i2 * 2 + b

            @pl.when(kk < n)
            def _():
                @pl.when(kk >= 2)
                def _():
                    wait_add(b)

                base = (start + kk) * CH
                pltpu.sync_copy(rowi_h.at[pl.ds(base, CH)], ivs[b])
                fill(base, b)
                pltpu.async_copy(bufs[b], acc.at[ivs[b]], sas[b], add=True)

        def run(fill):
            def body(i2, carry):
                slot(i2, 0, fill)
                slot(i2, 1, fill)
                return carry

            lax.fori_loop(0, trips, body, 0)
            for b in range(2):
                @pl.when(n > b)
                def _():
                    wait_add(b)

        @pl.when(c == 0)
        def _():
            def fill_m(base, b):
                pltpu.sync_copy(m_h.at[pl.ds(base, CH)], bufs[b])

            run(fill_m)

        @pl.when(c == 1)
        def _():
            def fill_rc(base, b):
                pltpu.sync_copy(rc_h.at[pl.ds(base, CH)], bufs[b])

            run(fill_rc)

        plsc.subcore_barrier()
        pltpu.sync_copy(acc.at[pl.ds(s * ZB, ZB)],
                        out_h.at[c, pl.ds(s * ZB, ZB)])

        @pl.when(s == 0)
        def _():
            pltpu.sync_copy(acc.at[pl.ds(NS * ZB, ZREM)],
                            out_h.at[c, pl.ds(NS * ZB, ZREM)])

    return k(m, rc128, rowi, zrows)


def _tc_prelude(time_step2, p):
    def body(ts_r, wt0_r, bt0_r, wt1_r, bt1_r, wtp_r, btp_r, t_out):
        half = H // 2
        i64f = lax.broadcasted_iota(jnp.int32, (1, half), 1).astype(jnp.float32)
        freqs = jnp.exp(-jnp.log(10000.0) * i64f / (half - 1))
        args = ts_r[...].astype(jnp.float32) * freqs
        temb = jnp.concatenate([jnp.sin(args), jnp.cos(args)], axis=1)
        t = jax.nn.relu(_mm(temb, wt0_r[...]) + bt0_r[...])
        t = jax.nn.relu(_mm(t, wt1_r[...]) + bt1_r[...])
        t_out[...] = _mm(t, wtp_r[...]) + btp_r[...]

    return pl.pallas_call(
        body,
        out_shape=jax.ShapeDtypeStruct((G, H), jnp.float32),
    )(time_step2, p["Wt0"], p["bt0"].reshape(1, -1), p["Wt1"],
      p["bt1"].reshape(1, -1), p["Wtp"], p["btp"].reshape(1, -1))


def _tc_node_prep(atom_type, batch2, t, p):
    def body(at_r, b_r, t_r, wa_r, wb_r, bin_r, out_r):
        gi = lax.broadcasted_iota(jnp.int32, (1, G), 1)
        oh = (b_r[...] == gi).astype(jnp.float32)
        tn = _mmsel(oh, t_r[...])
        out_r[...] = _mm(at_r[...], wa_r[...]) + _mm(tn, wb_r[...]) + bin_r[...]

    grid = (N // NBLK,)
    return pl.pallas_call(
        body,
        grid=grid,
        in_specs=[
            pl.BlockSpec((NBLK, H), lambda i: (i, 0)),
            pl.BlockSpec((NBLK, 1), lambda i: (i, 0)),
            pl.BlockSpec((G, H), lambda i: (0, 0)),
            pl.BlockSpec((H, H), lambda i: (0, 0)),
            pl.BlockSpec((H, H), lambda i: (0, 0)),
            pl.BlockSpec((1, H), lambda i: (0, 0)),
        ],
        out_specs=pl.BlockSpec((NBLK, H), lambda i: (i, 0)),
        out_shape=jax.ShapeDtypeStruct((N, H), jnp.float32),
    )(atom_type, batch2, t, p["Win"][:H], p["Win"][H:],
      p["bin"].reshape(1, -1))


def _ea_from_aux(aux, t_r, we1_r, be1_r, we2_r, be2_r, emb_r):
    el = aux[:, 5:6]
    etv = aux[:, 6:7]
    e2g = aux[:, 4:5]
    r1 = jax.nn.relu(el * we1_r[...] + be1_r[...])
    ea0 = _mm(r1, we2_r[...]) + be2_r[...]
    ei = lax.broadcasted_iota(jnp.int32, (1, 8), 1).astype(jnp.float32)
    ohe = (etv == ei).astype(jnp.float32)
    gi = lax.broadcasted_iota(jnp.int32, (1, G), 1).astype(jnp.float32)
    ohg = (e2g == gi).astype(jnp.float32)
    return ea0 * _mmsel(ohe, emb_r[...]) + _mmsel(ohg, t_r[...])


_EAW = None


def _ea_specs():
    W = pl.BlockSpec((H, H), lambda i: (0, 0))
    b1 = pl.BlockSpec((1, H), lambda i: (0, 0))
    return [
        pl.BlockSpec((G, H), lambda i: (0, 0)),
        pl.BlockSpec((1, H), lambda i: (0, 0)),
        b1, W, b1,
        pl.BlockSpec((8, H), lambda i: (0, 0)),
    ]


def _ea_args(t, p):
    return (t, p["We1"], p["be1"].reshape(1, -1), p["We2"],
            p["be2"].reshape(1, -1), p["emb_et"])


def _tc_edge_mlp(hr, hc, aux, t, p, lp):
    def body(hr_r, hc_r, aux_r, t_r, we1_r, be1_r, we2_r, be2_r, emb_r,
             w1r_r, w1c_r, w1d_r, w1e_r, bm1_r, wm2_r, bm2_r,
             wg_r, bg_r, wc1_r, wc2_r, m_out, rc_out):
        aux = aux_r[...]
        rel = aux[:, 0:3]
        d2 = aux[:, 3:4]
        ea = _ea_from_aux(aux, t_r, we1_r, be1_r, we2_r, be2_r, emb_r)
        d2b = d2.astype(jnp.bfloat16).astype(jnp.float32)
        w1db = w1d_r[...].astype(jnp.bfloat16).astype(jnp.float32)
        m = (_mm(hr_r[...], w1r_r[...]) + _mm(hc_r[...], w1c_r[...])
             + d2b * w1db + _mm(ea, w1e_r[...]) + bm1_r[...])
        m = _silu(m)
        m = _silu(_mm(m, wm2_r[...]) + bm2_r[...])
        m = m * jax.nn.sigmoid(_mm(m, wg_r[...]) + bg_r[...])
        coef = _mm(_silu(_mm(m, wc1_r[...])), wc2_r[...])
        reln = rel / (jnp.sqrt(d2) + 1.0)
        m_out[...] = m
        rc_out[...] = jnp.concatenate(
            [reln * coef, jnp.zeros((EBLK, H - 3), jnp.float32)], axis=1)

    ne = hr.shape[0]
    W = pl.BlockSpec((H, H), lambda i: (0, 0))
    b1 = pl.BlockSpec((1, H), lambda i: (0, 0))
    grid = (ne // EBLK,)
    return pl.pallas_call(
        body,
        grid=grid,
        in_specs=[
            pl.BlockSpec((EBLK, H), lambda i: (i, 0)),
            pl.BlockSpec((EBLK, H), lambda i: (i, 0)),
            pl.BlockSpec((EBLK, H), lambda i: (i, 0)),
        ] + _ea_specs() + [
            W, W, b1, W, b1, W, b1,
            pl.BlockSpec((H, 1), lambda i: (0, 0)),
            pl.BlockSpec((1, 1), lambda i: (0, 0)),
            W,
            pl.BlockSpec((H, 1), lambda i: (0, 0)),
        ],
        out_specs=(
            pl.BlockSpec((EBLK, H), lambda i: (i, 0)),
            pl.BlockSpec((EBLK, H), lambda i: (i, 0)),
        ),
        out_shape=(
            jax.ShapeDtypeStruct((ne, H), jnp.float32),
            jax.ShapeDtypeStruct((ne, H), jnp.float32),
        ),
    )(hr, hc, aux, *_ea_args(t, p),
      lp["Wm1"][:H], lp["Wm1"][H:2 * H], lp["Wm1"][2 * H:2 * H + 1],
      lp["Wm1"][2 * H + 1:], lp["bm1"].reshape(1, -1),
      lp["Wm2"], lp["bm2"].reshape(1, -1),
      lp["Wg"], lp["bg"].reshape(1, 1),
      lp["Wc1"], lp["Wc2"])


def _tc_node_update(table, x4, pA, pB, lp):
    def body(tbl_r, x4_r, pa_r, pb_r, whh_r, wha_r, bh1_r, wh2_r, bh2_r,
             h_out, x_out):
        pa = pa_r[...]
        pb = pb_r[...]
        agg = pa[0] + pb[0]
        h = tbl_r[...]
        upd = _mm(_silu(_mm(h, whh_r[...]) + _mm(agg, wha_r[...]) + bh1_r[...]),
                  wh2_r[...]) + bh2_r[...]
        h_out[...] = h + upd
        x_out[...] = x4_r[...] + (pa[1] + pb[1])[:, :4]

    W = pl.BlockSpec((H, H), lambda i: (0, 0))
    b1 = pl.BlockSpec((1, H), lambda i: (0, 0))
    grid = (N // NBLK,)
    return pl.pallas_call(
        body,
        grid=grid,
        in_specs=[
            pl.BlockSpec((NBLK, H), lambda i: (i, 0)),
            pl.BlockSpec((NBLK, 4), lambda i: (i, 0)),
            pl.BlockSpec((NC, NBLK, H), lambda i: (0, i, 0)),
            pl.BlockSpec((NC, NBLK, H), lambda i: (0, i, 0)),
            W, W, b1, W, b1,
        ],
        out_specs=(
            pl.BlockSpec((NBLK, H), lambda i: (i, 0)),
            pl.BlockSpec((NBLK, 4), lambda i: (i, 0)),
        ),
        out_shape=(
            jax.ShapeDtypeStruct((N, H), jnp.float32),
            jax.ShapeDtypeStruct((N, 4), jnp.float32),
        ),
    )(table, x4, pA, pB, lp["Wh1"][:H], lp["Wh1"][H:],
      lp["bh1"].reshape(1, -1), lp["Wh2"], lp["bh2"].reshape(1, -1))


def _tc_final(hr, hc, aux, t, p):
    def body(hr_r, hc_r, aux_r, t_r, we1_r, be1_r, we2_r, be2_r, emb_r,
             wd1h_r, wd1e_r, bd1_r, wd2_r, bd2_r, wd3t_r, bd3_r, out_r):
        ea = _ea_from_aux(aux_r[...], t_r, we1_r, be1_r, we2_r, be2_r, emb_r)
        g1 = jax.nn.relu(_mm(hr_r[...] * hc_r[...], wd1h_r[...])
                         + _mm(ea, wd1e_r[...]) + bd1_r[...])
        g2 = jax.nn.relu(_mm(g1, wd2_r[...]) + bd2_r[...])
        out_r[0] = _mm(wd3t_r[...], g2.T) + bd3_r[...]

    ne = hr.shape[0]
    grid = (ne // EBLK,)
    return pl.pallas_call(
        body,
        grid=grid,
        in_specs=[
            pl.BlockSpec((EBLK, H), lambda i: (i, 0)),
            pl.BlockSpec((EBLK, H), lambda i: (i, 0)),
            pl.BlockSpec((EBLK, H), lambda i: (i, 0)),
        ] + _ea_specs() + [
            pl.BlockSpec((H, H), lambda i: (0, 0)),
            pl.BlockSpec((H, H), lambda i: (0, 0)),
            pl.BlockSpec((1, H), lambda i: (0, 0)),
            pl.BlockSpec((H, H // 2), lambda i: (0, 0)),
            pl.BlockSpec((1, H // 2), lambda i: (0, 0)),
            pl.BlockSpec((1, H // 2), lambda i: (0, 0)),
            pl.BlockSpec((1, 1), lambda i: (0, 0)),
        ],
        out_specs=pl.BlockSpec((1, 1, EBLK), lambda i: (i, 0, 0)),
        out_shape=jax.ShapeDtypeStruct((ne // EBLK, 1, EBLK), jnp.float32),
    )(hr, hc, aux, *_ea_args(t, p),
      p["Wd1"][:H], p["Wd1"][H:], p["bd1"].reshape(1, -1),
      p["Wd2"], p["bd2"].reshape(1, -1), p["Wd3"].reshape(1, -1),
      p["bd3"].reshape(1, 1))


def kernel(atom_type, pos, bond_index, bond_type, batch, time_step, edge_type,
           edge_index, edge_length, params):
    p = params
    row = edge_index[0].astype(jnp.int32)
    col = edge_index[1].astype(jnp.int32)
    batchi = batch.astype(jnp.int32)
    batch2 = batchi.reshape(N, 1)
    ts2 = time_step.astype(jnp.int32).reshape(G, 1)
    eti = edge_type.astype(jnp.int32)
    elflat = edge_length.reshape(E)
    x4 = jnp.pad(pos, ((0, 0), (0, 1)))
    zrows = jnp.zeros((ZB, H), jnp.float32)

    t = _tc_prelude(ts2, p)
    table = _tc_node_prep(atom_type, batch2, t, p)

    E2 = E // 2
    halves = []
    for h0 in (0, E2):
        halves.append(dict(
            row=row[h0:h0 + E2], col=col[h0:h0 + E2],
            el=elflat[h0:h0 + E2], et=eti[h0:h0 + E2]))

    def gath(hv):
        return _sc_gather(table, x4.reshape(-1), hv["el"], hv["et"], batchi,
                          hv["row"], hv["col"], E2)

    for lp in p["layers"]:
        gA = gath(halves[0])
        gB = gath(halves[1])
        mA, rcA = _tc_edge_mlp(*gA, t, p, lp)
        sA = _sc_scatter(mA, rcA, halves[0]["row"], zrows, E2)
        mB, rcB = _tc_edge_mlp(*gB, t, p, lp)
        sB = _sc_scatter(mB, rcB, halves[1]["row"], zrows, E2)
        table, x4 = _tc_node_update(table, x4, sA, sB, lp)

    gA = gath(halves[0])
    gB = gath(halves[1])
    oA = _tc_final(*gA, t, p)
    oB = _tc_final(*gB, t, p)
    edge_inv = jnp.concatenate([oA, oB], axis=0).reshape(E, 1)

    local_edge_mask = edge_type == 0
    return (edge_inv, edge_index, edge_type, edge_length, local_edge_mask)


# speedup vs baseline: 1.0694x; 1.0685x over previous
import functools

import jax
import jax.numpy as jnp
from jax import lax
from jax.experimental import pallas as pl
from jax.experimental.pallas import tpu as pltpu
from jax.experimental.pallas import tpu_sc as plsc

N = 10000
E = 160000
G = 64
H = 128

EBLK = 1000
NBLK = 400

NC = 2
NS = 16
NW = NC * NS
L = 16
CH = 128
CHG = 64
ZB = 624
ZREM = N - NS * ZB


def _mesh():
    return plsc.VectorSubcoreMesh(
        core_axis_name="c", subcore_axis_name="s", num_cores=NC, num_subcores=NS
    )


def _silu(x):
    return x * jax.nn.sigmoid(x)


def _mm(a, b):
    return jax.lax.dot_general(
        a, b, (((1,), (0,)), ((), ())), preferred_element_type=jnp.float32
    )


def _mmsel(oh, table):
    hi = table.astype(jnp.bfloat16).astype(jnp.float32)
    lo = table - hi
    return _mm(oh, hi) + _mm(oh, lo)


def _worker_range(w, nper, nchunk):
    base = nchunk // nper
    rem = nchunk % nper
    start = w * base + jnp.minimum(w, rem)
    n = base + (w < rem).astype(jnp.int32)
    return start, n, (base + 2) // 2


def _sc_gather(table, x4flat, elflat, eti, batchi, rowi, coli, ne):
    @functools.partial(
        pl.kernel,
        out_type=(
            jax.ShapeDtypeStruct((ne, H), jnp.float32),
            jax.ShapeDtypeStruct((ne, H), jnp.float32),
            jax.ShapeDtypeStruct((ne, H), jnp.float32),
        ),
        mesh=_mesh(),
        scratch_types=[
            pltpu.VMEM((CHG,), jnp.int32),
            pltpu.VMEM((CHG,), jnp.int32),
            pltpu.VMEM((CHG,), jnp.int32),
            pltpu.VMEM((CHG,), jnp.int32),
            pltpu.VMEM((CHG,), jnp.float32),
            pltpu.VMEM((CHG,), jnp.float32),
            pltpu.VMEM((CHG,), jnp.int32),
            pltpu.VMEM((CHG,), jnp.int32),
            pltpu.VMEM((CHG, H), jnp.float32),
            pltpu.VMEM((CHG, H), jnp.float32),
            pltpu.VMEM((CHG, H), jnp.float32),
            pltpu.VMEM((CHG, H), jnp.float32),
            pltpu.VMEM((CHG, H), jnp.float32),
            pltpu.VMEM((CHG, H), jnp.float32),
            pltpu.VMEM((N * 4,), jnp.float32),
            pltpu.VMEM((N,), jnp.int32),
            pltpu.SemaphoreType.DMA,
            pltpu.SemaphoreType.DMA,
            pltpu.SemaphoreType.DMA,
            pltpu.SemaphoreType.DMA,
        ],
        compiler_params=pltpu.CompilerParams(needs_layout_passes=False),
    )
    def k(table_h, x4_h, el_h, et_h, b_h, rowi_h, coli_h,
          outr_h, outc_h, aux_h,
          ir0, ir1, ic0, ic1, elb0, elb1, etb0, etb1,
          br0, br1, bc0, bc1, ax0, ax1, x4v, bv,
          sr0, sr1, sc0, sc1):
        irs = (ir0, ir1)
        ics = (ic0, ic1)
        elbs = (elb0, elb1)
        etbs = (etb0, etb1)
        brs = (br0, br1)
        bcs = (bc0, bc1)
        axs = (ax0, ax1)
        srs = (sr0, sr1)
        scs = (sc0, sc1)
        c = lax.axis_index("c")
        s = lax.axis_index("s")
        w = s * NC + c
        start, n, trips = _worker_range(w, NW, ne // CHG)
        pltpu.sync_copy(x4_h, x4v)
        pltpu.sync_copy(b_h, bv)
        iota = lax.iota(jnp.int32, L)

        def load_and_fire(kk, b):
            base = (start + kk) * CHG
            pltpu.sync_copy(rowi_h.at[pl.ds(base, CHG)], irs[b])
            pltpu.sync_copy(coli_h.at[pl.ds(base, CHG)], ics[b])
            pltpu.sync_copy(el_h.at[pl.ds(base, CHG)], elbs[b])
            pltpu.sync_copy(et_h.at[pl.ds(base, CHG)], etbs[b])
            pltpu.async_copy(table_h.at[irs[b]], brs[b], srs[b])
            pltpu.async_copy(table_h.at[ics[b]], bcs[b], scs[b])

        for b in range(2):
            @pl.when(b < n)
            def _():
                load_and_fire(b, b)

        def slot(i2, b):
            kk = i2 * 2 + b

            @pl.when(kk < n)
            def _():
                base = (start + kk) * CHG
                for j in range(CHG // L):
                    sl = pl.ds(j * L, L)
                    rv = irs[b][sl]
                    cv = ics[b][sl]
                    rv4 = rv * 4
                    cv4 = cv * 4
                    erow = iota + j * L
                    rel = []
                    for nn in range(3):
                        rn = (plsc.load_gather(x4v, [rv4 + nn])
                              - plsc.load_gather(x4v, [cv4 + nn]))
                        rel.append(rn)
                        plsc.store_scatter(axs[b], [erow, jnp.full((L,), nn, jnp.int32)], rn)
                    d2 = rel[0] * rel[0] + rel[1] * rel[1] + rel[2] * rel[2]
                    plsc.store_scatter(axs[b], [erow, jnp.full((L,), 3, jnp.int32)], d2)
                    e2g = plsc.load_gather(bv, [rv]).astype(jnp.float32)
                    plsc.store_scatter(axs[b], [erow, jnp.full((L,), 4, jnp.int32)], e2g)
                    plsc.store_scatter(axs[b], [erow, jnp.full((L,), 5, jnp.int32)], elbs[b][sl])
                    plsc.store_scatter(axs[b], [erow, jnp.full((L,), 6, jnp.int32)],
                                       etbs[b][sl].astype(jnp.float32))
                pltpu.make_async_copy(table_h.at[irs[b]], brs[b],
                                      srs[b]).wait()
                pltpu.make_async_copy(table_h.at[ics[b]], bcs[b],
                                      scs[b]).wait()
                pltpu.sync_copy(brs[b], outr_h.at[pl.ds(base, CHG)])
                pltpu.sync_copy(bcs[b], outc_h.at[pl.ds(base, CHG)])
                pltpu.sync_copy(axs[b], aux_h.at[pl.ds(base, CHG)])

                @pl.when(kk + 2 < n)
                def _():
                    load_and_fire(kk + 2, b)

        def body(i2, carry):
            slot(i2, 0)
            slot(i2, 1)
            return carry

        lax.fori_loop(0, trips, body, 0)

    return k(table, x4flat, elflat, eti, batchi, rowi, coli)


def _sc_scatter(m, rc128, rowi, zrows, ne):
    @functools.partial(
        pl.kernel,
        out_type=jax.ShapeDtypeStruct((NC, N, H), jnp.float32),
        mesh=_mesh(),
        scratch_types=[
            pltpu.VMEM_SHARED((N, H), jnp.float32),
            pltpu.VMEM((CH, H), jnp.float32),
            pltpu.VMEM((CH, H), jnp.float32),
            pltpu.VMEM((CH,), jnp.int32),
            pltpu.VMEM((CH,), jnp.int32),
            pltpu.SemaphoreType.DMA,
            pltpu.SemaphoreType.DMA,
        ],
    )
    def k(m_h, rc_h, rowi_h, z_h, out_h, acc, buf0, buf1, iv0, iv1, sa0, sa1):
        bufs = (buf0, buf1)
        ivs = (iv0, iv1)
        sas = (sa0, sa1)
        c = lax.axis_index("c")
        s = lax.axis_index("s")
        pltpu.sync_copy(z_h, acc.at[pl.ds(s * ZB, ZB)])

        @pl.when(s == 0)
        def _():
            pltpu.sync_copy(z_h.at[pl.ds(0, ZREM)], acc.at[pl.ds(NS * ZB, ZREM)])

        plsc.subcore_barrier()
        start, n, trips = _worker_range(s, NS, ne // CH)

        def wait_add(b):
            pltpu.make_async_copy(bufs[b], acc.at[ivs[b]], sas[b]).wait()

        def slot(i2, b, fill):
            kk = i2 * 2 + b

            @pl.when(kk < n)
            def _():
                @pl.when(kk >= 2)
                def _():
                    wait_add(b)

                base = (start + kk) * CH
                pltpu.sync_copy(rowi_h.at[pl.ds(base, CH)], ivs[b])
                fill(base, b)
                pltpu.async_copy(bufs[b], acc.at[ivs[b]], sas[b], add=True)

        def run(fill):
            def body(i2, carry):
                slot(i2, 0, fill)
                slot(i2, 1, fill)
                return carry

            lax.fori_loop(0, trips, body, 0)
            for b in range(2):
                @pl.when(n > b)
                def _():
                    wait_add(b)

        @pl.when(c == 0)
        def _():
            def fill_m(base, b):
                pltpu.sync_copy(m_h.at[pl.ds(base, CH)], bufs[b])

            run(fill_m)

        @pl.when(c == 1)
        def _():
            def fill_rc(base, b):
                pltpu.sync_copy(rc_h.at[pl.ds(base, CH)], bufs[b])

            run(fill_rc)

        plsc.subcore_barrier()
        pltpu.sync_copy(acc.at[pl.ds(s * ZB, ZB)],
                        out_h.at[c, pl.ds(s * ZB, ZB)])

        @pl.when(s == 0)
        def _():
            pltpu.sync_copy(acc.at[pl.ds(NS * ZB, ZREM)],
                            out_h.at[c, pl.ds(NS * ZB, ZREM)])

    return k(m, rc128, rowi, zrows)


def _tc_prelude(time_step2, p):
    def body(ts_r, wt0_r, bt0_r, wt1_r, bt1_r, wtp_r, btp_r, t_out):
        half = H // 2
        i64f = lax.broadcasted_iota(jnp.int32, (1, half), 1).astype(jnp.float32)
        freqs = jnp.exp(-jnp.log(10000.0) * i64f / (half - 1))
        args = ts_r[...].astype(jnp.float32) * freqs
        temb = jnp.concatenate([jnp.sin(args), jnp.cos(args)], axis=1)
        t = jax.nn.relu(_mm(temb, wt0_r[...]) + bt0_r[...])
        t = jax.nn.relu(_mm(t, wt1_r[...]) + bt1_r[...])
        t_out[...] = _mm(t, wtp_r[...]) + btp_r[...]

    return pl.pallas_call(
        body,
        out_shape=jax.ShapeDtypeStruct((G, H), jnp.float32),
    )(time_step2, p["Wt0"], p["bt0"].reshape(1, -1), p["Wt1"],
      p["bt1"].reshape(1, -1), p["Wtp"], p["btp"].reshape(1, -1))


def _tc_node_prep(atom_type, batch2, t, p):
    def body(at_r, b_r, t_r, wa_r, wb_r, bin_r, out_r):
        gi = lax.broadcasted_iota(jnp.int32, (1, G), 1)
        oh = (b_r[...] == gi).astype(jnp.float32)
        tn = _mmsel(oh, t_r[...])
        out_r[...] = _mm(at_r[...], wa_r[...]) + _mm(tn, wb_r[...]) + bin_r[...]

    grid = (N // NBLK,)
    return pl.pallas_call(
        body,
        grid=grid,
        in_specs=[
            pl.BlockSpec((NBLK, H), lambda i: (i, 0)),
            pl.BlockSpec((NBLK, 1), lambda i: (i, 0)),
            pl.BlockSpec((G, H), lambda i: (0, 0)),
            pl.BlockSpec((H, H), lambda i: (0, 0)),
            pl.BlockSpec((H, H), lambda i: (0, 0)),
            pl.BlockSpec((1, H), lambda i: (0, 0)),
        ],
        out_specs=pl.BlockSpec((NBLK, H), lambda i: (i, 0)),
        out_shape=jax.ShapeDtypeStruct((N, H), jnp.float32),
    )(atom_type, batch2, t, p["Win"][:H], p["Win"][H:],
      p["bin"].reshape(1, -1))


def _ea_from_aux(aux, t_r, we1_r, be1_r, we2_r, be2_r, emb_r):
    el = aux[:, 5:6]
    etv = aux[:, 6:7]
    e2g = aux[:, 4:5]
    r1 = jax.nn.relu(el * we1_r[...] + be1_r[...])
    ea0 = _mm(r1, we2_r[...]) + be2_r[...]
    ei = lax.broadcasted_iota(jnp.int32, (1, 8), 1).astype(jnp.float32)
    ohe = (etv == ei).astype(jnp.float32)
    gi = lax.broadcasted_iota(jnp.int32, (1, G), 1).astype(jnp.float32)
    ohg = (e2g == gi).astype(jnp.float32)
    return ea0 * _mmsel(ohe, emb_r[...]) + _mmsel(ohg, t_r[...])


_EAW = None


def _ea_specs():
    W = pl.BlockSpec((H, H), lambda i: (0, 0))
    b1 = pl.BlockSpec((1, H), lambda i: (0, 0))
    return [
        pl.BlockSpec((G, H), lambda i: (0, 0)),
        pl.BlockSpec((1, H), lambda i: (0, 0)),
        b1, W, b1,
        pl.BlockSpec((8, H), lambda i: (0, 0)),
    ]


def _ea_args(t, p):
    return (t, p["We1"], p["be1"].reshape(1, -1), p["We2"],
            p["be2"].reshape(1, -1), p["emb_et"])


def _tc_edge_mlp(hr, hc, aux, t, p, lp):
    def body(hr_r, hc_r, aux_r, t_r, we1_r, be1_r, we2_r, be2_r, emb_r,
             w1r_r, w1c_r, w1d_r, w1e_r, bm1_r, wm2_r, bm2_r,
             wg_r, bg_r, wc1_r, wc2_r, m_out, rc_out):
        aux = aux_r[...]
        rel = aux[:, 0:3]
        d2 = aux[:, 3:4]
        ea = _ea_from_aux(aux, t_r, we1_r, be1_r, we2_r, be2_r, emb_r)
        d2b = d2.astype(jnp.bfloat16).astype(jnp.float32)
        w1db = w1d_r[...].astype(jnp.bfloat16).astype(jnp.float32)
        m = (_mm(hr_r[...], w1r_r[...]) + _mm(hc_r[...], w1c_r[...])
             + d2b * w1db + _mm(ea, w1e_r[...]) + bm1_r[...])
        m = _silu(m)
        m = _silu(_mm(m, wm2_r[...]) + bm2_r[...])
        m = m * jax.nn.sigmoid(_mm(m, wg_r[...]) + bg_r[...])
        coef = _mm(_silu(_mm(m, wc1_r[...])), wc2_r[...])
        reln = rel / (jnp.sqrt(d2) + 1.0)
        m_out[...] = m
        rc_out[...] = jnp.concatenate(
            [reln * coef, jnp.zeros((EBLK, H - 3), jnp.float32)], axis=1)

    ne = hr.shape[0]
    W = pl.BlockSpec((H, H), lambda i: (0, 0))
    b1 = pl.BlockSpec((1, H), lambda i: (0, 0))
    grid = (ne // EBLK,)
    return pl.pallas_call(
        body,
        grid=grid,
        in_specs=[
            pl.BlockSpec((EBLK, H), lambda i: (i, 0)),
            pl.BlockSpec((EBLK, H), lambda i: (i, 0)),
            pl.BlockSpec((EBLK, H), lambda i: (i, 0)),
        ] + _ea_specs() + [
            W, W, b1, W, b1, W, b1,
            pl.BlockSpec((H, 1), lambda i: (0, 0)),
            pl.BlockSpec((1, 1), lambda i: (0, 0)),
            W,
            pl.BlockSpec((H, 1), lambda i: (0, 0)),
        ],
        out_specs=(
            pl.BlockSpec((EBLK, H), lambda i: (i, 0)),
            pl.BlockSpec((EBLK, H), lambda i: (i, 0)),
        ),
        out_shape=(
            jax.ShapeDtypeStruct((ne, H), jnp.float32),
            jax.ShapeDtypeStruct((ne, H), jnp.float32),
        ),
    )(hr, hc, aux, *_ea_args(t, p),
      lp["Wm1"][:H], lp["Wm1"][H:2 * H], lp["Wm1"][2 * H:2 * H + 1],
      lp["Wm1"][2 * H + 1:], lp["bm1"].reshape(1, -1),
      lp["Wm2"], lp["bm2"].reshape(1, -1),
      lp["Wg"], lp["bg"].reshape(1, 1),
      lp["Wc1"], lp["Wc2"])


def _tc_node_update(table, x4, pA, pB, lp):
    def body(tbl_r, x4_r, pa_r, pb_r, whh_r, wha_r, bh1_r, wh2_r, bh2_r,
             h_out, x_out):
        pa = pa_r[...]
        pb = pb_r[...]
        agg = pa[0] + pb[0]
        h = tbl_r[...]
        upd = _mm(_silu(_mm(h, whh_r[...]) + _mm(agg, wha_r[...]) + bh1_r[...]),
                  wh2_r[...]) + bh2_r[...]
        h_out[...] = h + upd
        x_out[...] = x4_r[...] + (pa[1] + pb[1])[:, :4]

    W = pl.BlockSpec((H, H), lambda i: (0, 0))
    b1 = pl.BlockSpec((1, H), lambda i: (0, 0))
    grid = (N // NBLK,)
    return pl.pallas_call(
        body,
        grid=grid,
        in_specs=[
            pl.BlockSpec((NBLK, H), lambda i: (i, 0)),
            pl.BlockSpec((NBLK, 4), lambda i: (i, 0)),
            pl.BlockSpec((NC, NBLK, H), lambda i: (0, i, 0)),
            pl.BlockSpec((NC, NBLK, H), lambda i: (0, i, 0)),
            W, W, b1, W, b1,
        ],
        out_specs=(
            pl.BlockSpec((NBLK, H), lambda i: (i, 0)),
            pl.BlockSpec((NBLK, 4), lambda i: (i, 0)),
        ),
        out_shape=(
            jax.ShapeDtypeStruct((N, H), jnp.float32),
            jax.ShapeDtypeStruct((N, 4), jnp.float32),
        ),
    )(table, x4, pA, pB, lp["Wh1"][:H], lp["Wh1"][H:],
      lp["bh1"].reshape(1, -1), lp["Wh2"], lp["bh2"].reshape(1, -1))


def _tc_final(hr, hc, aux, t, p):
    def body(hr_r, hc_r, aux_r, t_r, we1_r, be1_r, we2_r, be2_r, emb_r,
             wd1h_r, wd1e_r, bd1_r, wd2_r, bd2_r, wd3t_r, bd3_r, out_r):
        ea = _ea_from_aux(aux_r[...], t_r, we1_r, be1_r, we2_r, be2_r, emb_r)
        g1 = jax.nn.relu(_mm(hr_r[...] * hc_r[...], wd1h_r[...])
                         + _mm(ea, wd1e_r[...]) + bd1_r[...])
        g2 = jax.nn.relu(_mm(g1, wd2_r[...]) + bd2_r[...])
        out_r[0] = _mm(wd3t_r[...], g2.T) + bd3_r[...]

    ne = hr.shape[0]
    grid = (ne // EBLK,)
    return pl.pallas_call(
        body,
        grid=grid,
        in_specs=[
            pl.BlockSpec((EBLK, H), lambda i: (i, 0)),
            pl.BlockSpec((EBLK, H), lambda i: (i, 0)),
            pl.BlockSpec((EBLK, H), lambda i: (i, 0)),
        ] + _ea_specs() + [
            pl.BlockSpec((H, H), lambda i: (0, 0)),
            pl.BlockSpec((H, H), lambda i: (0, 0)),
            pl.BlockSpec((1, H), lambda i: (0, 0)),
            pl.BlockSpec((H, H // 2), lambda i: (0, 0)),
            pl.BlockSpec((1, H // 2), lambda i: (0, 0)),
            pl.BlockSpec((1, H // 2), lambda i: (0, 0)),
            pl.BlockSpec((1, 1), lambda i: (0, 0)),
        ],
        out_specs=pl.BlockSpec((1, 1, EBLK), lambda i: (i, 0, 0)),
        out_shape=jax.ShapeDtypeStruct((ne // EBLK, 1, EBLK), jnp.float32),
    )(hr, hc, aux, *_ea_args(t, p),
      p["Wd1"][:H], p["Wd1"][H:], p["bd1"].reshape(1, -1),
      p["Wd2"], p["bd2"].reshape(1, -1), p["Wd3"].reshape(1, -1),
      p["bd3"].reshape(1, 1))


def kernel(atom_type, pos, bond_index, bond_type, batch, time_step, edge_type,
           edge_index, edge_length, params):
    p = params
    row = edge_index[0].astype(jnp.int32)
    col = edge_index[1].astype(jnp.int32)
    batchi = batch.astype(jnp.int32)
    batch2 = batchi.reshape(N, 1)
    ts2 = time_step.astype(jnp.int32).reshape(G, 1)
    eti = edge_type.astype(jnp.int32)
    elflat = edge_length.reshape(E)
    x4 = jnp.pad(pos, ((0, 0), (0, 1)))
    zrows = jnp.zeros((ZB, H), jnp.float32)

    t = _tc_prelude(ts2, p)
    table = _tc_node_prep(atom_type, batch2, t, p)

    E2 = E // 2
    halves = []
    for h0 in (0, E2):
        halves.append(dict(
            row=row[h0:h0 + E2], col=col[h0:h0 + E2],
            el=elflat[h0:h0 + E2], et=eti[h0:h0 + E2]))

    def gath(hv):
        return _sc_gather(table, x4.reshape(-1), hv["el"], hv["et"], batchi,
                          hv["row"], hv["col"], E2)

    for lp in p["layers"]:
        gA = gath(halves[0])
        gB = gath(halves[1])
        mA, rcA = _tc_edge_mlp(*gA, t, p, lp)
        sA = _sc_scatter(mA, rcA, halves[0]["row"], zrows, E2)
        mB, rcB = _tc_edge_mlp(*gB, t, p, lp)
        sB = _sc_scatter(mB, rcB, halves[1]["row"], zrows, E2)
        table, x4 = _tc_node_update(table, x4, sA, sB, lp)

    gA = gath(halves[0])
    gB = gath(halves[1])
    oA = _tc_final(*gA, t, p)
    oB = _tc_final(*gB, t, p)
    edge_inv = jnp.concatenate([oA, oB], axis=0).reshape(E, 1)

    local_edge_mask = edge_type == 0
    return (edge_inv, edge_index, edge_type, edge_length, local_edge_mask)


# speedup vs baseline: 1.1951x; 1.1176x over previous
import functools

import jax
import jax.numpy as jnp
from jax import lax
from jax.experimental import pallas as pl
from jax.experimental.pallas import tpu as pltpu
from jax.experimental.pallas import tpu_sc as plsc

N = 10000
E = 160000
G = 64
H = 128

EBLK = 2000
NBLK = 1000

NC = 2
NS = 16
NW = NC * NS
L = 16
CH = 128
CHG = 64
ZB = 624
ZREM = N - NS * ZB


def _mesh():
    return plsc.VectorSubcoreMesh(
        core_axis_name="c", subcore_axis_name="s", num_cores=NC, num_subcores=NS
    )


def _silu(x):
    return x * jax.nn.sigmoid(x)


def _mm(a, b):
    return jax.lax.dot_general(
        a, b, (((1,), (0,)), ((), ())), preferred_element_type=jnp.float32
    )


def _mmsel(oh, table):
    hi = table.astype(jnp.bfloat16).astype(jnp.float32)
    lo = table - hi
    return _mm(oh, hi) + _mm(oh, lo)


def _worker_range(w, nper, nchunk):
    base = nchunk // nper
    rem = nchunk % nper
    start = w * base + jnp.minimum(w, rem)
    n = base + (w < rem).astype(jnp.int32)
    return start, n, (base + 2) // 2


def _sc_gather(table, x4flat, elflat, eti, batchi, rowi, coli, ne):
    @functools.partial(
        pl.kernel,
        out_type=(
            jax.ShapeDtypeStruct((ne, H), jnp.float32),
            jax.ShapeDtypeStruct((ne, H), jnp.float32),
            jax.ShapeDtypeStruct((ne, H), jnp.float32),
        ),
        mesh=_mesh(),
        scratch_types=[
            pltpu.VMEM((CHG,), jnp.int32),
            pltpu.VMEM((CHG,), jnp.int32),
            pltpu.VMEM((CHG,), jnp.int32),
            pltpu.VMEM((CHG,), jnp.int32),
            pltpu.VMEM((CHG,), jnp.float32),
            pltpu.VMEM((CHG,), jnp.float32),
            pltpu.VMEM((CHG,), jnp.int32),
            pltpu.VMEM((CHG,), jnp.int32),
            pltpu.VMEM((CHG, H), jnp.float32),
            pltpu.VMEM((CHG, H), jnp.float32),
            pltpu.VMEM((CHG, H), jnp.float32),
            pltpu.VMEM((CHG, H), jnp.float32),
            pltpu.VMEM((CHG, H), jnp.float32),
            pltpu.VMEM((CHG, H), jnp.float32),
            pltpu.VMEM((N * 4,), jnp.float32),
            pltpu.VMEM((N,), jnp.int32),
            pltpu.SemaphoreType.DMA,
            pltpu.SemaphoreType.DMA,
            pltpu.SemaphoreType.DMA,
            pltpu.SemaphoreType.DMA,
        ],
        compiler_params=pltpu.CompilerParams(needs_layout_passes=False),
    )
    def k(table_h, x4_h, el_h, et_h, b_h, rowi_h, coli_h,
          outr_h, outc_h, aux_h,
          ir0, ir1, ic0, ic1, elb0, elb1, etb0, etb1,
          br0, br1, bc0, bc1, ax0, ax1, x4v, bv,
          sr0, sr1, sc0, sc1):
        irs = (ir0, ir1)
        ics = (ic0, ic1)
        elbs = (elb0, elb1)
        etbs = (etb0, etb1)
        brs = (br0, br1)
        bcs = (bc0, bc1)
        axs = (ax0, ax1)
        srs = (sr0, sr1)
        scs = (sc0, sc1)
        c = lax.axis_index("c")
        s = lax.axis_index("s")
        w = s * NC + c
        start, n, trips = _worker_range(w, NW, ne // CHG)
        pltpu.sync_copy(x4_h, x4v)
        pltpu.sync_copy(b_h, bv)
        iota = lax.iota(jnp.int32, L)

        def load_and_fire(kk, b):
            base = (start + kk) * CHG
            pltpu.sync_copy(rowi_h.at[pl.ds(base, CHG)], irs[b])
            pltpu.sync_copy(coli_h.at[pl.ds(base, CHG)], ics[b])
            pltpu.sync_copy(el_h.at[pl.ds(base, CHG)], elbs[b])
            pltpu.sync_copy(et_h.at[pl.ds(base, CHG)], etbs[b])
            pltpu.async_copy(table_h.at[irs[b]], brs[b], srs[b])
            pltpu.async_copy(table_h.at[ics[b]], bcs[b], scs[b])

        for b in range(2):
            @pl.when(b < n)
            def _():
                load_and_fire(b, b)

        def slot(i2, b):
            kk = i2 * 2 + b

            @pl.when(kk < n)
            def _():
                base = (start + kk) * CHG
                for j in range(CHG // L):
                    sl = pl.ds(j * L, L)
                    rv = irs[b][sl]
                    cv = ics[b][sl]
                    rv4 = rv * 4
                    cv4 = cv * 4
                    erow = iota + j * L
                    rel = []
                    for nn in range(3):
                        rn = (plsc.load_gather(x4v, [rv4 + nn])
                              - plsc.load_gather(x4v, [cv4 + nn]))
                        rel.append(rn)
                        plsc.store_scatter(axs[b], [erow, jnp.full((L,), nn, jnp.int32)], rn)
                    d2 = rel[0] * rel[0] + rel[1] * rel[1] + rel[2] * rel[2]
                    plsc.store_scatter(axs[b], [erow, jnp.full((L,), 3, jnp.int32)], d2)
                    e2g = plsc.load_gather(bv, [rv]).astype(jnp.float32)
                    plsc.store_scatter(axs[b], [erow, jnp.full((L,), 4, jnp.int32)], e2g)
                    plsc.store_scatter(axs[b], [erow, jnp.full((L,), 5, jnp.int32)], elbs[b][sl])
                    plsc.store_scatter(axs[b], [erow, jnp.full((L,), 6, jnp.int32)],
                                       etbs[b][sl].astype(jnp.float32))
                pltpu.make_async_copy(table_h.at[irs[b]], brs[b],
                                      srs[b]).wait()
                pltpu.make_async_copy(table_h.at[ics[b]], bcs[b],
                                      scs[b]).wait()
                pltpu.sync_copy(brs[b], outr_h.at[pl.ds(base, CHG)])
                pltpu.sync_copy(bcs[b], outc_h.at[pl.ds(base, CHG)])
                pltpu.sync_copy(axs[b], aux_h.at[pl.ds(base, CHG)])

                @pl.when(kk + 2 < n)
                def _():
                    load_and_fire(kk + 2, b)

        def body(i2, carry):
            slot(i2, 0)
            slot(i2, 1)
            return carry

        lax.fori_loop(0, trips, body, 0)

    return k(table, x4flat, elflat, eti, batchi, rowi, coli)


def _sc_scatter(m, rc128, rowi, zrows, ne):
    @functools.partial(
        pl.kernel,
        out_type=jax.ShapeDtypeStruct((NC, N, H), jnp.float32),
        mesh=_mesh(),
        scratch_types=[
            pltpu.VMEM_SHARED((N, H), jnp.float32),
            pltpu.VMEM((CH, H), jnp.float32),
            pltpu.VMEM((CH, H), jnp.float32),
            pltpu.VMEM((CH,), jnp.int32),
            pltpu.VMEM((CH,), jnp.int32),
            pltpu.SemaphoreType.DMA,
            pltpu.SemaphoreType.DMA,
        ],
    )
    def k(m_h, rc_h, rowi_h, z_h, out_h, acc, buf0, buf1, iv0, iv1, sa0, sa1):
        bufs = (buf0, buf1)
        ivs = (iv0, iv1)
        sas = (sa0, sa1)
        c = lax.axis_index("c")
        s = lax.axis_index("s")
        pltpu.sync_copy(z_h, acc.at[pl.ds(s * ZB, ZB)])

        @pl.when(s == 0)
        def _():
            pltpu.sync_copy(z_h.at[pl.ds(0, ZREM)], acc.at[pl.ds(NS * ZB, ZREM)])

        plsc.subcore_barrier()
        start, n, trips = _worker_range(s, NS, ne // CH)

        def wait_add(b):
            pltpu.make_async_copy(bufs[b], acc.at[ivs[b]], sas[b]).wait()

        def slot(i2, b, fill):
            kk = i2 * 2 + b

            @pl.when(kk < n)
            def _():
                @pl.when(kk >= 2)
                def _():
                    wait_add(b)

                base = (start + kk) * CH
                pltpu.sync_copy(rowi_h.at[pl.ds(base, CH)], ivs[b])
                fill(base, b)
                pltpu.async_copy(bufs[b], acc.at[ivs[b]], sas[b], add=True)

        def run(fill):
            def body(i2, carry):
                slot(i2, 0, fill)
                slot(i2, 1, fill)
                return carry

            lax.fori_loop(0, trips, body, 0)
            for b in range(2):
                @pl.when(n > b)
                def _():
                    wait_add(b)

        @pl.when(c == 0)
        def _():
            def fill_m(base, b):
                pltpu.sync_copy(m_h.at[pl.ds(base, CH)], bufs[b])

            run(fill_m)

        @pl.when(c == 1)
        def _():
            def fill_rc(base, b):
                pltpu.sync_copy(rc_h.at[pl.ds(base, CH)], bufs[b])

            run(fill_rc)

        plsc.subcore_barrier()
        pltpu.sync_copy(acc.at[pl.ds(s * ZB, ZB)],
                        out_h.at[c, pl.ds(s * ZB, ZB)])

        @pl.when(s == 0)
        def _():
            pltpu.sync_copy(acc.at[pl.ds(NS * ZB, ZREM)],
                            out_h.at[c, pl.ds(NS * ZB, ZREM)])

    return k(m, rc128, rowi, zrows)


def _tc_prelude(time_step2, p):
    def body(ts_r, wt0_r, bt0_r, wt1_r, bt1_r, wtp_r, btp_r, t_out):
        half = H // 2
        i64f = lax.broadcasted_iota(jnp.int32, (1, half), 1).astype(jnp.float32)
        freqs = jnp.exp(-jnp.log(10000.0) * i64f / (half - 1))
        args = ts_r[...].astype(jnp.float32) * freqs
        temb = jnp.concatenate([jnp.sin(args), jnp.cos(args)], axis=1)
        t = jax.nn.relu(_mm(temb, wt0_r[...]) + bt0_r[...])
        t = jax.nn.relu(_mm(t, wt1_r[...]) + bt1_r[...])
        t_out[...] = _mm(t, wtp_r[...]) + btp_r[...]

    return pl.pallas_call(
        body,
        out_shape=jax.ShapeDtypeStruct((G, H), jnp.float32),
    )(time_step2, p["Wt0"], p["bt0"].reshape(1, -1), p["Wt1"],
      p["bt1"].reshape(1, -1), p["Wtp"], p["btp"].reshape(1, -1))


def _tc_node_prep(atom_type, batch2, t, p):
    def body(at_r, b_r, t_r, wa_r, wb_r, bin_r, out_r):
        gi = lax.broadcasted_iota(jnp.int32, (1, G), 1)
        oh = (b_r[...] == gi).astype(jnp.float32)
        tn = _mmsel(oh, t_r[...])
        out_r[...] = _mm(at_r[...], wa_r[...]) + _mm(tn, wb_r[...]) + bin_r[...]

    grid = (N // NBLK,)
    return pl.pallas_call(
        body,
        grid=grid,
        in_specs=[
            pl.BlockSpec((NBLK, H), lambda i: (i, 0)),
            pl.BlockSpec((NBLK, 1), lambda i: (i, 0)),
            pl.BlockSpec((G, H), lambda i: (0, 0)),
            pl.BlockSpec((H, H), lambda i: (0, 0)),
            pl.BlockSpec((H, H), lambda i: (0, 0)),
            pl.BlockSpec((1, H), lambda i: (0, 0)),
        ],
        out_specs=pl.BlockSpec((NBLK, H), lambda i: (i, 0)),
        out_shape=jax.ShapeDtypeStruct((N, H), jnp.float32),
    )(atom_type, batch2, t, p["Win"][:H], p["Win"][H:],
      p["bin"].reshape(1, -1))


def _ea_from_aux(aux, t_r, we1_r, be1_r, we2_r, be2_r, emb_r):
    el = aux[:, 5:6]
    etv = aux[:, 6:7]
    e2g = aux[:, 4:5]
    r1 = jax.nn.relu(el * we1_r[...] + be1_r[...])
    ea0 = _mm(r1, we2_r[...]) + be2_r[...]
    ei = lax.broadcasted_iota(jnp.int32, (1, 8), 1).astype(jnp.float32)
    ohe = (etv == ei).astype(jnp.float32)
    gi = lax.broadcasted_iota(jnp.int32, (1, G), 1).astype(jnp.float32)
    ohg = (e2g == gi).astype(jnp.float32)
    return ea0 * _mmsel(ohe, emb_r[...]) + _mmsel(ohg, t_r[...])


_EAW = None


def _ea_specs():
    W = pl.BlockSpec((H, H), lambda i: (0, 0))
    b1 = pl.BlockSpec((1, H), lambda i: (0, 0))
    return [
        pl.BlockSpec((G, H), lambda i: (0, 0)),
        pl.BlockSpec((1, H), lambda i: (0, 0)),
        b1, W, b1,
        pl.BlockSpec((8, H), lambda i: (0, 0)),
    ]


def _ea_args(t, p):
    return (t, p["We1"], p["be1"].reshape(1, -1), p["We2"],
            p["be2"].reshape(1, -1), p["emb_et"])


def _tc_edge_mlp(hr, hc, aux, t, p, lp):
    def body(hr_r, hc_r, aux_r, t_r, we1_r, be1_r, we2_r, be2_r, emb_r,
             w1r_r, w1c_r, w1d_r, w1e_r, bm1_r, wm2_r, bm2_r,
             wg_r, bg_r, wc1_r, wc2_r, m_out, rc_out):
        aux = aux_r[...]
        rel = aux[:, 0:3]
        d2 = aux[:, 3:4]
        ea = _ea_from_aux(aux, t_r, we1_r, be1_r, we2_r, be2_r, emb_r)
        d2b = d2.astype(jnp.bfloat16).astype(jnp.float32)
        w1db = w1d_r[...].astype(jnp.bfloat16).astype(jnp.float32)
        m = (_mm(hr_r[...], w1r_r[...]) + _mm(hc_r[...], w1c_r[...])
             + d2b * w1db + _mm(ea, w1e_r[...]) + bm1_r[...])
        m = _silu(m)
        m = _silu(_mm(m, wm2_r[...]) + bm2_r[...])
        m = m * jax.nn.sigmoid(_mm(m, wg_r[...]) + bg_r[...])
        coef = _mm(_silu(_mm(m, wc1_r[...])), wc2_r[...])
        reln = rel / (jnp.sqrt(d2) + 1.0)
        m_out[...] = m
        rc_out[...] = jnp.concatenate(
            [reln * coef, jnp.zeros((EBLK, H - 3), jnp.float32)], axis=1)

    ne = hr.shape[0]
    W = pl.BlockSpec((H, H), lambda i: (0, 0))
    b1 = pl.BlockSpec((1, H), lambda i: (0, 0))
    grid = (ne // EBLK,)
    return pl.pallas_call(
        body,
        grid=grid,
        in_specs=[
            pl.BlockSpec((EBLK, H), lambda i: (i, 0)),
            pl.BlockSpec((EBLK, H), lambda i: (i, 0)),
            pl.BlockSpec((EBLK, H), lambda i: (i, 0)),
        ] + _ea_specs() + [
            W, W, b1, W, b1, W, b1,
            pl.BlockSpec((H, 1), lambda i: (0, 0)),
            pl.BlockSpec((1, 1), lambda i: (0, 0)),
            W,
            pl.BlockSpec((H, 1), lambda i: (0, 0)),
        ],
        out_specs=(
            pl.BlockSpec((EBLK, H), lambda i: (i, 0)),
            pl.BlockSpec((EBLK, H), lambda i: (i, 0)),
        ),
        out_shape=(
            jax.ShapeDtypeStruct((ne, H), jnp.float32),
            jax.ShapeDtypeStruct((ne, H), jnp.float32),
        ),
    )(hr, hc, aux, *_ea_args(t, p),
      lp["Wm1"][:H], lp["Wm1"][H:2 * H], lp["Wm1"][2 * H:2 * H + 1],
      lp["Wm1"][2 * H + 1:], lp["bm1"].reshape(1, -1),
      lp["Wm2"], lp["bm2"].reshape(1, -1),
      lp["Wg"], lp["bg"].reshape(1, 1),
      lp["Wc1"], lp["Wc2"])


def _tc_node_update(table, x4, pA, pB, lp):
    def body(tbl_r, x4_r, pa_r, pb_r, whh_r, wha_r, bh1_r, wh2_r, bh2_r,
             h_out, x_out):
        pa = pa_r[...]
        pb = pb_r[...]
        agg = pa[0] + pb[0]
        h = tbl_r[...]
        upd = _mm(_silu(_mm(h, whh_r[...]) + _mm(agg, wha_r[...]) + bh1_r[...]),
                  wh2_r[...]) + bh2_r[...]
        h_out[...] = h + upd
        x_out[...] = x4_r[...] + (pa[1] + pb[1])[:, :4]

    W = pl.BlockSpec((H, H), lambda i: (0, 0))
    b1 = pl.BlockSpec((1, H), lambda i: (0, 0))
    grid = (N // NBLK,)
    return pl.pallas_call(
        body,
        grid=grid,
        in_specs=[
            pl.BlockSpec((NBLK, H), lambda i: (i, 0)),
            pl.BlockSpec((NBLK, 4), lambda i: (i, 0)),
            pl.BlockSpec((NC, NBLK, H), lambda i: (0, i, 0)),
            pl.BlockSpec((NC, NBLK, H), lambda i: (0, i, 0)),
            W, W, b1, W, b1,
        ],
        out_specs=(
            pl.BlockSpec((NBLK, H), lambda i: (i, 0)),
            pl.BlockSpec((NBLK, 4), lambda i: (i, 0)),
        ),
        out_shape=(
            jax.ShapeDtypeStruct((N, H), jnp.float32),
            jax.ShapeDtypeStruct((N, 4), jnp.float32),
        ),
    )(table, x4, pA, pB, lp["Wh1"][:H], lp["Wh1"][H:],
      lp["bh1"].reshape(1, -1), lp["Wh2"], lp["bh2"].reshape(1, -1))


def _tc_final(hr, hc, aux, t, p):
    def body(hr_r, hc_r, aux_r, t_r, we1_r, be1_r, we2_r, be2_r, emb_r,
             wd1h_r, wd1e_r, bd1_r, wd2_r, bd2_r, wd3t_r, bd3_r, out_r):
        ea = _ea_from_aux(aux_r[...], t_r, we1_r, be1_r, we2_r, be2_r, emb_r)
        g1 = jax.nn.relu(_mm(hr_r[...] * hc_r[...], wd1h_r[...])
                         + _mm(ea, wd1e_r[...]) + bd1_r[...])
        g2 = jax.nn.relu(_mm(g1, wd2_r[...]) + bd2_r[...])
        out_r[0] = _mm(wd3t_r[...], g2.T) + bd3_r[...]

    ne = hr.shape[0]
    grid = (ne // EBLK,)
    return pl.pallas_call(
        body,
        grid=grid,
        in_specs=[
            pl.BlockSpec((EBLK, H), lambda i: (i, 0)),
            pl.BlockSpec((EBLK, H), lambda i: (i, 0)),
            pl.BlockSpec((EBLK, H), lambda i: (i, 0)),
        ] + _ea_specs() + [
            pl.BlockSpec((H, H), lambda i: (0, 0)),
            pl.BlockSpec((H, H), lambda i: (0, 0)),
            pl.BlockSpec((1, H), lambda i: (0, 0)),
            pl.BlockSpec((H, H // 2), lambda i: (0, 0)),
            pl.BlockSpec((1, H // 2), lambda i: (0, 0)),
            pl.BlockSpec((1, H // 2), lambda i: (0, 0)),
            pl.BlockSpec((1, 1), lambda i: (0, 0)),
        ],
        out_specs=pl.BlockSpec((1, 1, EBLK), lambda i: (i, 0, 0)),
        out_shape=jax.ShapeDtypeStruct((ne // EBLK, 1, EBLK), jnp.float32),
    )(hr, hc, aux, *_ea_args(t, p),
      p["Wd1"][:H], p["Wd1"][H:], p["bd1"].reshape(1, -1),
      p["Wd2"], p["bd2"].reshape(1, -1), p["Wd3"].reshape(1, -1),
      p["bd3"].reshape(1, 1))


def kernel(atom_type, pos, bond_index, bond_type, batch, time_step, edge_type,
           edge_index, edge_length, params):
    p = params
    row = edge_index[0].astype(jnp.int32)
    col = edge_index[1].astype(jnp.int32)
    batchi = batch.astype(jnp.int32)
    batch2 = batchi.reshape(N, 1)
    ts2 = time_step.astype(jnp.int32).reshape(G, 1)
    eti = edge_type.astype(jnp.int32)
    elflat = edge_length.reshape(E)
    x4 = jnp.pad(pos, ((0, 0), (0, 1)))
    zrows = jnp.zeros((ZB, H), jnp.float32)

    t = _tc_prelude(ts2, p)
    table = _tc_node_prep(atom_type, batch2, t, p)

    E2 = E // 2
    halves = []
    for h0 in (0, E2):
        halves.append(dict(
            row=row[h0:h0 + E2], col=col[h0:h0 + E2],
            el=elflat[h0:h0 + E2], et=eti[h0:h0 + E2]))

    def gath(hv):
        return _sc_gather(table, x4.reshape(-1), hv["el"], hv["et"], batchi,
                          hv["row"], hv["col"], E2)

    for lp in p["layers"]:
        gA = gath(halves[0])
        gB = gath(halves[1])
        mA, rcA = _tc_edge_mlp(*gA, t, p, lp)
        sA = _sc_scatter(mA, rcA, halves[0]["row"], zrows, E2)
        mB, rcB = _tc_edge_mlp(*gB, t, p, lp)
        sB = _sc_scatter(mB, rcB, halves[1]["row"], zrows, E2)
        table, x4 = _tc_node_update(table, x4, sA, sB, lp)

    gA = gath(halves[0])
    gB = gath(halves[1])
    oA = _tc_final(*gA, t, p)
    oB = _tc_final(*gB, t, p)
    edge_inv = jnp.concatenate([oA, oB], axis=0).reshape(E, 1)

    local_edge_mask = edge_type == 0
    return (edge_inv, edge_index, edge_type, edge_length, local_edge_mask)


# speedup vs baseline: 1.2111x; 1.0134x over previous
import functools

import jax
import jax.numpy as jnp
from jax import lax
from jax.experimental import pallas as pl
from jax.experimental.pallas import tpu as pltpu
from jax.experimental.pallas import tpu_sc as plsc

N = 10000
E = 160000
G = 64
H = 128

EBLK = 4000
NBLK = 2000

NC = 2
NS = 16
NW = NC * NS
L = 16
CH = 128
CHG = 64
ZB = 624
ZREM = N - NS * ZB


def _mesh():
    return plsc.VectorSubcoreMesh(
        core_axis_name="c", subcore_axis_name="s", num_cores=NC, num_subcores=NS
    )


def _silu(x):
    return x * jax.nn.sigmoid(x)


def _mm(a, b):
    return jax.lax.dot_general(
        a, b, (((1,), (0,)), ((), ())), preferred_element_type=jnp.float32
    )


def _mmsel(oh, table):
    hi = table.astype(jnp.bfloat16).astype(jnp.float32)
    lo = table - hi
    return _mm(oh, hi) + _mm(oh, lo)


def _worker_range(w, nper, nchunk):
    base = nchunk // nper
    rem = nchunk % nper
    start = w * base + jnp.minimum(w, rem)
    n = base + (w < rem).astype(jnp.int32)
    return start, n, (base + 2) // 2


def _sc_gather(table, x4flat, elflat, eti, batchi, rowi, coli, ne):
    @functools.partial(
        pl.kernel,
        out_type=(
            jax.ShapeDtypeStruct((ne, H), jnp.float32),
            jax.ShapeDtypeStruct((ne, H), jnp.float32),
            jax.ShapeDtypeStruct((ne, H), jnp.float32),
        ),
        mesh=_mesh(),
        scratch_types=[
            pltpu.VMEM((CHG,), jnp.int32),
            pltpu.VMEM((CHG,), jnp.int32),
            pltpu.VMEM((CHG,), jnp.int32),
            pltpu.VMEM((CHG,), jnp.int32),
            pltpu.VMEM((CHG,), jnp.float32),
            pltpu.VMEM((CHG,), jnp.float32),
            pltpu.VMEM((CHG,), jnp.int32),
            pltpu.VMEM((CHG,), jnp.int32),
            pltpu.VMEM((CHG, H), jnp.float32),
            pltpu.VMEM((CHG, H), jnp.float32),
            pltpu.VMEM((CHG, H), jnp.float32),
            pltpu.VMEM((CHG, H), jnp.float32),
            pltpu.VMEM((CHG, H), jnp.float32),
            pltpu.VMEM((CHG, H), jnp.float32),
            pltpu.VMEM((N * 4,), jnp.float32),
            pltpu.VMEM((N,), jnp.int32),
            pltpu.SemaphoreType.DMA,
            pltpu.SemaphoreType.DMA,
            pltpu.SemaphoreType.DMA,
            pltpu.SemaphoreType.DMA,
        ],
        compiler_params=pltpu.CompilerParams(needs_layout_passes=False),
    )
    def k(table_h, x4_h, el_h, et_h, b_h, rowi_h, coli_h,
          outr_h, outc_h, aux_h,
          ir0, ir1, ic0, ic1, elb0, elb1, etb0, etb1,
          br0, br1, bc0, bc1, ax0, ax1, x4v, bv,
          sr0, sr1, sc0, sc1):
        irs = (ir0, ir1)
        ics = (ic0, ic1)
        elbs = (elb0, elb1)
        etbs = (etb0, etb1)
        brs = (br0, br1)
        bcs = (bc0, bc1)
        axs = (ax0, ax1)
        srs = (sr0, sr1)
        scs = (sc0, sc1)
        c = lax.axis_index("c")
        s = lax.axis_index("s")
        w = s * NC + c
        start, n, trips = _worker_range(w, NW, ne // CHG)
        pltpu.sync_copy(x4_h, x4v)
        pltpu.sync_copy(b_h, bv)
        iota = lax.iota(jnp.int32, L)

        def load_and_fire(kk, b):
            base = (start + kk) * CHG
            pltpu.sync_copy(rowi_h.at[pl.ds(base, CHG)], irs[b])
            pltpu.sync_copy(coli_h.at[pl.ds(base, CHG)], ics[b])
            pltpu.sync_copy(el_h.at[pl.ds(base, CHG)], elbs[b])
            pltpu.sync_copy(et_h.at[pl.ds(base, CHG)], etbs[b])
            pltpu.async_copy(table_h.at[irs[b]], brs[b], srs[b])
            pltpu.async_copy(table_h.at[ics[b]], bcs[b], scs[b])

        for b in range(2):
            @pl.when(b < n)
            def _():
                load_and_fire(b, b)

        def slot(i2, b):
            kk = i2 * 2 + b

            @pl.when(kk < n)
            def _():
                base = (start + kk) * CHG
                for j in range(CHG // L):
                    sl = pl.ds(j * L, L)
                    rv = irs[b][sl]
                    cv = ics[b][sl]
                    rv4 = rv * 4
                    cv4 = cv * 4
                    erow = iota + j * L
                    rel = []
                    for nn in range(3):
                        rn = (plsc.load_gather(x4v, [rv4 + nn])
                              - plsc.load_gather(x4v, [cv4 + nn]))
                        rel.append(rn)
                        plsc.store_scatter(axs[b], [erow, jnp.full((L,), nn, jnp.int32)], rn)
                    d2 = rel[0] * rel[0] + rel[1] * rel[1] + rel[2] * rel[2]
                    plsc.store_scatter(axs[b], [erow, jnp.full((L,), 3, jnp.int32)], d2)
                    e2g = plsc.load_gather(bv, [rv]).astype(jnp.float32)
                    plsc.store_scatter(axs[b], [erow, jnp.full((L,), 4, jnp.int32)], e2g)
                    plsc.store_scatter(axs[b], [erow, jnp.full((L,), 5, jnp.int32)], elbs[b][sl])
                    plsc.store_scatter(axs[b], [erow, jnp.full((L,), 6, jnp.int32)],
                                       etbs[b][sl].astype(jnp.float32))
                pltpu.make_async_copy(table_h.at[irs[b]], brs[b],
                                      srs[b]).wait()
                pltpu.make_async_copy(table_h.at[ics[b]], bcs[b],
                                      scs[b]).wait()
                pltpu.sync_copy(brs[b], outr_h.at[pl.ds(base, CHG)])
                pltpu.sync_copy(bcs[b], outc_h.at[pl.ds(base, CHG)])
                pltpu.sync_copy(axs[b], aux_h.at[pl.ds(base, CHG)])

                @pl.when(kk + 2 < n)
                def _():
                    load_and_fire(kk + 2, b)

        def body(i2, carry):
            slot(i2, 0)
            slot(i2, 1)
            return carry

        lax.fori_loop(0, trips, body, 0)

    return k(table, x4flat, elflat, eti, batchi, rowi, coli)


def _sc_scatter(m, rc128, rowi, zrows, ne):
    @functools.partial(
        pl.kernel,
        out_type=jax.ShapeDtypeStruct((NC, N, H), jnp.float32),
        mesh=_mesh(),
        scratch_types=[
            pltpu.VMEM_SHARED((N, H), jnp.float32),
            pltpu.VMEM((CH, H), jnp.float32),
            pltpu.VMEM((CH, H), jnp.float32),
            pltpu.VMEM((CH,), jnp.int32),
            pltpu.VMEM((CH,), jnp.int32),
            pltpu.SemaphoreType.DMA,
            pltpu.SemaphoreType.DMA,
        ],
    )
    def k(m_h, rc_h, rowi_h, z_h, out_h, acc, buf0, buf1, iv0, iv1, sa0, sa1):
        bufs = (buf0, buf1)
        ivs = (iv0, iv1)
        sas = (sa0, sa1)
        c = lax.axis_index("c")
        s = lax.axis_index("s")
        pltpu.sync_copy(z_h, acc.at[pl.ds(s * ZB, ZB)])

        @pl.when(s == 0)
        def _():
            pltpu.sync_copy(z_h.at[pl.ds(0, ZREM)], acc.at[pl.ds(NS * ZB, ZREM)])

        plsc.subcore_barrier()
        start, n, trips = _worker_range(s, NS, ne // CH)

        def wait_add(b):
            pltpu.make_async_copy(bufs[b], acc.at[ivs[b]], sas[b]).wait()

        def slot(i2, b, fill):
            kk = i2 * 2 + b

            @pl.when(kk < n)
            def _():
                @pl.when(kk >= 2)
                def _():
                    wait_add(b)

                base = (start + kk) * CH
                pltpu.sync_copy(rowi_h.at[pl.ds(base, CH)], ivs[b])
                fill(base, b)
                pltpu.async_copy(bufs[b], acc.at[ivs[b]], sas[b], add=True)

        def run(fill):
            def body(i2, carry):
                slot(i2, 0, fill)
                slot(i2, 1, fill)
                return carry

            lax.fori_loop(0, trips, body, 0)
            for b in range(2):
                @pl.when(n > b)
                def _():
                    wait_add(b)

        @pl.when(c == 0)
        def _():
            def fill_m(base, b):
                pltpu.sync_copy(m_h.at[pl.ds(base, CH)], bufs[b])

            run(fill_m)

        @pl.when(c == 1)
        def _():
            def fill_rc(base, b):
                pltpu.sync_copy(rc_h.at[pl.ds(base, CH)], bufs[b])

            run(fill_rc)

        plsc.subcore_barrier()
        pltpu.sync_copy(acc.at[pl.ds(s * ZB, ZB)],
                        out_h.at[c, pl.ds(s * ZB, ZB)])

        @pl.when(s == 0)
        def _():
            pltpu.sync_copy(acc.at[pl.ds(NS * ZB, ZREM)],
                            out_h.at[c, pl.ds(NS * ZB, ZREM)])

    return k(m, rc128, rowi, zrows)


def _tc_prelude(time_step2, p):
    def body(ts_r, wt0_r, bt0_r, wt1_r, bt1_r, wtp_r, btp_r, t_out):
        half = H // 2
        i64f = lax.broadcasted_iota(jnp.int32, (1, half), 1).astype(jnp.float32)
        freqs = jnp.exp(-jnp.log(10000.0) * i64f / (half - 1))
        args = ts_r[...].astype(jnp.float32) * freqs
        temb = jnp.concatenate([jnp.sin(args), jnp.cos(args)], axis=1)
        t = jax.nn.relu(_mm(temb, wt0_r[...]) + bt0_r[...])
        t = jax.nn.relu(_mm(t, wt1_r[...]) + bt1_r[...])
        t_out[...] = _mm(t, wtp_r[...]) + btp_r[...]

    return pl.pallas_call(
        body,
        out_shape=jax.ShapeDtypeStruct((G, H), jnp.float32),
    )(time_step2, p["Wt0"], p["bt0"].reshape(1, -1), p["Wt1"],
      p["bt1"].reshape(1, -1), p["Wtp"], p["btp"].reshape(1, -1))


def _tc_node_prep(atom_type, batch2, t, p):
    def body(at_r, b_r, t_r, wa_r, wb_r, bin_r, out_r):
        gi = lax.broadcasted_iota(jnp.int32, (1, G), 1)
        oh = (b_r[...] == gi).astype(jnp.float32)
        tn = _mmsel(oh, t_r[...])
        out_r[...] = _mm(at_r[...], wa_r[...]) + _mm(tn, wb_r[...]) + bin_r[...]

    grid = (N // NBLK,)
    return pl.pallas_call(
        body,
        grid=grid,
        in_specs=[
            pl.BlockSpec((NBLK, H), lambda i: (i, 0)),
            pl.BlockSpec((NBLK, 1), lambda i: (i, 0)),
            pl.BlockSpec((G, H), lambda i: (0, 0)),
            pl.BlockSpec((H, H), lambda i: (0, 0)),
            pl.BlockSpec((H, H), lambda i: (0, 0)),
            pl.BlockSpec((1, H), lambda i: (0, 0)),
        ],
        out_specs=pl.BlockSpec((NBLK, H), lambda i: (i, 0)),
        out_shape=jax.ShapeDtypeStruct((N, H), jnp.float32),
    )(atom_type, batch2, t, p["Win"][:H], p["Win"][H:],
      p["bin"].reshape(1, -1))


def _ea_from_aux(aux, t_r, we1_r, be1_r, we2_r, be2_r, emb_r):
    el = aux[:, 5:6]
    etv = aux[:, 6:7]
    e2g = aux[:, 4:5]
    r1 = jax.nn.relu(el * we1_r[...] + be1_r[...])
    ea0 = _mm(r1, we2_r[...]) + be2_r[...]
    ei = lax.broadcasted_iota(jnp.int32, (1, 8), 1).astype(jnp.float32)
    ohe = (etv == ei).astype(jnp.float32)
    gi = lax.broadcasted_iota(jnp.int32, (1, G), 1).astype(jnp.float32)
    ohg = (e2g == gi).astype(jnp.float32)
    return ea0 * _mmsel(ohe, emb_r[...]) + _mmsel(ohg, t_r[...])


_EAW = None


def _ea_specs():
    W = pl.BlockSpec((H, H), lambda i: (0, 0))
    b1 = pl.BlockSpec((1, H), lambda i: (0, 0))
    return [
        pl.BlockSpec((G, H), lambda i: (0, 0)),
        pl.BlockSpec((1, H), lambda i: (0, 0)),
        b1, W, b1,
        pl.BlockSpec((8, H), lambda i: (0, 0)),
    ]


def _ea_args(t, p):
    return (t, p["We1"], p["be1"].reshape(1, -1), p["We2"],
            p["be2"].reshape(1, -1), p["emb_et"])


def _tc_edge_mlp(hr, hc, aux, t, p, lp):
    def body(hr_r, hc_r, aux_r, t_r, we1_r, be1_r, we2_r, be2_r, emb_r,
             w1r_r, w1c_r, w1d_r, w1e_r, bm1_r, wm2_r, bm2_r,
             wg_r, bg_r, wc1_r, wc2_r, m_out, rc_out):
        aux = aux_r[...]
        rel = aux[:, 0:3]
        d2 = aux[:, 3:4]
        ea = _ea_from_aux(aux, t_r, we1_r, be1_r, we2_r, be2_r, emb_r)
        d2b = d2.astype(jnp.bfloat16).astype(jnp.float32)
        w1db = w1d_r[...].astype(jnp.bfloat16).astype(jnp.float32)
        m = (_mm(hr_r[...], w1r_r[...]) + _mm(hc_r[...], w1c_r[...])
             + d2b * w1db + _mm(ea, w1e_r[...]) + bm1_r[...])
        m = _silu(m)
        m = _silu(_mm(m, wm2_r[...]) + bm2_r[...])
        m = m * jax.nn.sigmoid(_mm(m, wg_r[...]) + bg_r[...])
        coef = _mm(_silu(_mm(m, wc1_r[...])), wc2_r[...])
        reln = rel / (jnp.sqrt(d2) + 1.0)
        m_out[...] = m
        rc_out[...] = jnp.concatenate(
            [reln * coef, jnp.zeros((EBLK, H - 3), jnp.float32)], axis=1)

    ne = hr.shape[0]
    W = pl.BlockSpec((H, H), lambda i: (0, 0))
    b1 = pl.BlockSpec((1, H), lambda i: (0, 0))
    grid = (ne // EBLK,)
    return pl.pallas_call(
        body,
        grid=grid,
        in_specs=[
            pl.BlockSpec((EBLK, H), lambda i: (i, 0)),
            pl.BlockSpec((EBLK, H), lambda i: (i, 0)),
            pl.BlockSpec((EBLK, H), lambda i: (i, 0)),
        ] + _ea_specs() + [
            W, W, b1, W, b1, W, b1,
            pl.BlockSpec((H, 1), lambda i: (0, 0)),
            pl.BlockSpec((1, 1), lambda i: (0, 0)),
            W,
            pl.BlockSpec((H, 1), lambda i: (0, 0)),
        ],
        out_specs=(
            pl.BlockSpec((EBLK, H), lambda i: (i, 0)),
            pl.BlockSpec((EBLK, H), lambda i: (i, 0)),
        ),
        out_shape=(
            jax.ShapeDtypeStruct((ne, H), jnp.float32),
            jax.ShapeDtypeStruct((ne, H), jnp.float32),
        ),
    )(hr, hc, aux, *_ea_args(t, p),
      lp["Wm1"][:H], lp["Wm1"][H:2 * H], lp["Wm1"][2 * H:2 * H + 1],
      lp["Wm1"][2 * H + 1:], lp["bm1"].reshape(1, -1),
      lp["Wm2"], lp["bm2"].reshape(1, -1),
      lp["Wg"], lp["bg"].reshape(1, 1),
      lp["Wc1"], lp["Wc2"])


def _tc_node_update(table, x4, pA, pB, lp):
    def body(tbl_r, x4_r, pa_r, pb_r, whh_r, wha_r, bh1_r, wh2_r, bh2_r,
             h_out, x_out):
        pa = pa_r[...]
        pb = pb_r[...]
        agg = pa[0] + pb[0]
        h = tbl_r[...]
        upd = _mm(_silu(_mm(h, whh_r[...]) + _mm(agg, wha_r[...]) + bh1_r[...]),
                  wh2_r[...]) + bh2_r[...]
        h_out[...] = h + upd
        x_out[...] = x4_r[...] + (pa[1] + pb[1])[:, :4]

    W = pl.BlockSpec((H, H), lambda i: (0, 0))
    b1 = pl.BlockSpec((1, H), lambda i: (0, 0))
    grid = (N // NBLK,)
    return pl.pallas_call(
        body,
        grid=grid,
        in_specs=[
            pl.BlockSpec((NBLK, H), lambda i: (i, 0)),
            pl.BlockSpec((NBLK, 4), lambda i: (i, 0)),
            pl.BlockSpec((NC, NBLK, H), lambda i: (0, i, 0)),
            pl.BlockSpec((NC, NBLK, H), lambda i: (0, i, 0)),
            W, W, b1, W, b1,
        ],
        out_specs=(
            pl.BlockSpec((NBLK, H), lambda i: (i, 0)),
            pl.BlockSpec((NBLK, 4), lambda i: (i, 0)),
        ),
        out_shape=(
            jax.ShapeDtypeStruct((N, H), jnp.float32),
            jax.ShapeDtypeStruct((N, 4), jnp.float32),
        ),
    )(table, x4, pA, pB, lp["Wh1"][:H], lp["Wh1"][H:],
      lp["bh1"].reshape(1, -1), lp["Wh2"], lp["bh2"].reshape(1, -1))


def _tc_final(hr, hc, aux, t, p):
    def body(hr_r, hc_r, aux_r, t_r, we1_r, be1_r, we2_r, be2_r, emb_r,
             wd1h_r, wd1e_r, bd1_r, wd2_r, bd2_r, wd3t_r, bd3_r, out_r):
        ea = _ea_from_aux(aux_r[...], t_r, we1_r, be1_r, we2_r, be2_r, emb_r)
        g1 = jax.nn.relu(_mm(hr_r[...] * hc_r[...], wd1h_r[...])
                         + _mm(ea, wd1e_r[...]) + bd1_r[...])
        g2 = jax.nn.relu(_mm(g1, wd2_r[...]) + bd2_r[...])
        out_r[0] = _mm(wd3t_r[...], g2.T) + bd3_r[...]

    ne = hr.shape[0]
    grid = (ne // EBLK,)
    return pl.pallas_call(
        body,
        grid=grid,
        in_specs=[
            pl.BlockSpec((EBLK, H), lambda i: (i, 0)),
            pl.BlockSpec((EBLK, H), lambda i: (i, 0)),
            pl.BlockSpec((EBLK, H), lambda i: (i, 0)),
        ] + _ea_specs() + [
            pl.BlockSpec((H, H), lambda i: (0, 0)),
            pl.BlockSpec((H, H), lambda i: (0, 0)),
            pl.BlockSpec((1, H), lambda i: (0, 0)),
            pl.BlockSpec((H, H // 2), lambda i: (0, 0)),
            pl.BlockSpec((1, H // 2), lambda i: (0, 0)),
            pl.BlockSpec((1, H // 2), lambda i: (0, 0)),
            pl.BlockSpec((1, 1), lambda i: (0, 0)),
        ],
        out_specs=pl.BlockSpec((1, 1, EBLK), lambda i: (i, 0, 0)),
        out_shape=jax.ShapeDtypeStruct((ne // EBLK, 1, EBLK), jnp.float32),
    )(hr, hc, aux, *_ea_args(t, p),
      p["Wd1"][:H], p["Wd1"][H:], p["bd1"].reshape(1, -1),
      p["Wd2"], p["bd2"].reshape(1, -1), p["Wd3"].reshape(1, -1),
      p["bd3"].reshape(1, 1))


def kernel(atom_type, pos, bond_index, bond_type, batch, time_step, edge_type,
           edge_index, edge_length, params):
    p = params
    row = edge_index[0].astype(jnp.int32)
    col = edge_index[1].astype(jnp.int32)
    batchi = batch.astype(jnp.int32)
    batch2 = batchi.reshape(N, 1)
    ts2 = time_step.astype(jnp.int32).reshape(G, 1)
    eti = edge_type.astype(jnp.int32)
    elflat = edge_length.reshape(E)
    x4 = jnp.pad(pos, ((0, 0), (0, 1)))
    zrows = jnp.zeros((ZB, H), jnp.float32)

    t = _tc_prelude(ts2, p)
    table = _tc_node_prep(atom_type, batch2, t, p)

    E2 = E // 2
    halves = []
    for h0 in (0, E2):
        halves.append(dict(
            row=row[h0:h0 + E2], col=col[h0:h0 + E2],
            el=elflat[h0:h0 + E2], et=eti[h0:h0 + E2]))

    def gath(hv):
        return _sc_gather(table, x4.reshape(-1), hv["el"], hv["et"], batchi,
                          hv["row"], hv["col"], E2)

    for lp in p["layers"]:
        gA = gath(halves[0])
        gB = gath(halves[1])
        mA, rcA = _tc_edge_mlp(*gA, t, p, lp)
        sA = _sc_scatter(mA, rcA, halves[0]["row"], zrows, E2)
        mB, rcB = _tc_edge_mlp(*gB, t, p, lp)
        sB = _sc_scatter(mB, rcB, halves[1]["row"], zrows, E2)
        table, x4 = _tc_node_update(table, x4, sA, sB, lp)

    gA = gath(halves[0])
    gB = gath(halves[1])
    oA = _tc_final(*gA, t, p)
    oB = _tc_final(*gB, t, p)
    edge_inv = jnp.concatenate([oA, oB], axis=0).reshape(E, 1)

    local_edge_mask = edge_type == 0
    return (edge_inv, edge_index, edge_type, edge_length, local_edge_mask)


# speedup vs baseline: 1.3297x; 1.0979x over previous
import functools

import jax
import jax.numpy as jnp
from jax import lax
from jax.experimental import pallas as pl
from jax.experimental.pallas import tpu as pltpu
from jax.experimental.pallas import tpu_sc as plsc

N = 10000
E = 160000
G = 64
H = 128

EBLK = 4000
NBLK = 2000

NC = 2
NS = 16
NW = NC * NS
L = 16
CH = 128
CHG = 64
ZB = 624
ZREM = N - NS * ZB


def _mesh():
    return plsc.VectorSubcoreMesh(
        core_axis_name="c", subcore_axis_name="s", num_cores=NC, num_subcores=NS
    )


def _silu(x):
    return x * jax.nn.sigmoid(x)


def _mm(a, b):
    return jax.lax.dot_general(
        a, b, (((1,), (0,)), ((), ())), preferred_element_type=jnp.float32
    )


def _mmsel(oh, table):
    hi = table.astype(jnp.bfloat16).astype(jnp.float32)
    lo = table - hi
    return _mm(oh, hi) + _mm(oh, lo)


def _worker_range(w, nper, nchunk):
    base = nchunk // nper
    rem = nchunk % nper
    start = w * base + jnp.minimum(w, rem)
    n = base + (w < rem).astype(jnp.int32)
    return start, n, (base + 2) // 2


def _sc_gather(table, x4flat, elflat, eti, batchi, rowi, coli, ne):
    @functools.partial(
        pl.kernel,
        out_type=(
            jax.ShapeDtypeStruct((ne, H), jnp.float32),
            jax.ShapeDtypeStruct((ne, H), jnp.float32),
            jax.ShapeDtypeStruct((ne, H), jnp.float32),
        ),
        mesh=_mesh(),
        scratch_types=[
            pltpu.VMEM((CHG,), jnp.int32),
            pltpu.VMEM((CHG,), jnp.int32),
            pltpu.VMEM((CHG,), jnp.int32),
            pltpu.VMEM((CHG,), jnp.int32),
            pltpu.VMEM((CHG,), jnp.float32),
            pltpu.VMEM((CHG,), jnp.float32),
            pltpu.VMEM((CHG,), jnp.int32),
            pltpu.VMEM((CHG,), jnp.int32),
            pltpu.VMEM((CHG, H), jnp.float32),
            pltpu.VMEM((CHG, H), jnp.float32),
            pltpu.VMEM((CHG, H), jnp.float32),
            pltpu.VMEM((CHG, H), jnp.float32),
            pltpu.VMEM((CHG, H), jnp.float32),
            pltpu.VMEM((CHG, H), jnp.float32),
            pltpu.VMEM((N * 4,), jnp.float32),
            pltpu.VMEM((N,), jnp.int32),
            pltpu.SemaphoreType.DMA,
            pltpu.SemaphoreType.DMA,
            pltpu.SemaphoreType.DMA,
            pltpu.SemaphoreType.DMA,
            pltpu.SemaphoreType.DMA,
            pltpu.SemaphoreType.DMA,
            pltpu.SemaphoreType.DMA,
            pltpu.SemaphoreType.DMA,
            pltpu.SemaphoreType.DMA,
            pltpu.SemaphoreType.DMA,
        ],
        compiler_params=pltpu.CompilerParams(needs_layout_passes=False),
    )
    def k(table_h, x4_h, el_h, et_h, b_h, rowi_h, coli_h,
          outr_h, outc_h, aux_h,
          ir0, ir1, ic0, ic1, elb0, elb1, etb0, etb1,
          br0, br1, bc0, bc1, ax0, ax1, x4v, bv,
          sr0, sr1, sc0, sc1, wr0, wr1, wc0, wc1, wa0, wa1):
        irs = (ir0, ir1)
        ics = (ic0, ic1)
        elbs = (elb0, elb1)
        etbs = (etb0, etb1)
        brs = (br0, br1)
        bcs = (bc0, bc1)
        axs = (ax0, ax1)
        srs = (sr0, sr1)
        scs = (sc0, sc1)
        wrs = (wr0, wr1)
        wcs = (wc0, wc1)
        was = (wa0, wa1)
        c = lax.axis_index("c")
        s = lax.axis_index("s")
        w = s * NC + c
        start, n, trips = _worker_range(w, NW, ne // CHG)
        pltpu.sync_copy(x4_h, x4v)
        pltpu.sync_copy(b_h, bv)
        iota = lax.iota(jnp.int32, L)

        def load_and_fire(kk, b):
            base = (start + kk) * CHG
            pltpu.sync_copy(rowi_h.at[pl.ds(base, CHG)], irs[b])
            pltpu.sync_copy(coli_h.at[pl.ds(base, CHG)], ics[b])
            pltpu.sync_copy(el_h.at[pl.ds(base, CHG)], elbs[b])
            pltpu.sync_copy(et_h.at[pl.ds(base, CHG)], etbs[b])
            pltpu.async_copy(table_h.at[irs[b]], brs[b], srs[b])
            pltpu.async_copy(table_h.at[ics[b]], bcs[b], scs[b])

        for b in range(2):
            @pl.when(b < n)
            def _():
                load_and_fire(b, b)

        def wait_rc(b, base):
            pltpu.make_async_copy(brs[b], outr_h.at[pl.ds(base, CHG)],
                                  wrs[b]).wait()
            pltpu.make_async_copy(bcs[b], outc_h.at[pl.ds(base, CHG)],
                                  wcs[b]).wait()

        def wait_aux(b, base):
            pltpu.make_async_copy(axs[b], aux_h.at[pl.ds(base, CHG)],
                                  was[b]).wait()

        def slot(i2, b):
            kk = i2 * 2 + b

            @pl.when(kk < n)
            def _():
                base = (start + kk) * CHG

                @pl.when(kk >= 2)
                def _():
                    wait_aux(b, (start + kk - 2) * CHG)

                for j in range(CHG // L):
                    sl = pl.ds(j * L, L)
                    rv = irs[b][sl]
                    cv = ics[b][sl]
                    rv4 = rv * 4
                    cv4 = cv * 4
                    erow = iota + j * L
                    rel = []
                    for nn in range(3):
                        rn = (plsc.load_gather(x4v, [rv4 + nn])
                              - plsc.load_gather(x4v, [cv4 + nn]))
                        rel.append(rn)
                        plsc.store_scatter(axs[b], [erow, jnp.full((L,), nn, jnp.int32)], rn)
                    d2 = rel[0] * rel[0] + rel[1] * rel[1] + rel[2] * rel[2]
                    plsc.store_scatter(axs[b], [erow, jnp.full((L,), 3, jnp.int32)], d2)
                    e2g = plsc.load_gather(bv, [rv]).astype(jnp.float32)
                    plsc.store_scatter(axs[b], [erow, jnp.full((L,), 4, jnp.int32)], e2g)
                    plsc.store_scatter(axs[b], [erow, jnp.full((L,), 5, jnp.int32)], elbs[b][sl])
                    plsc.store_scatter(axs[b], [erow, jnp.full((L,), 6, jnp.int32)],
                                       etbs[b][sl].astype(jnp.float32))
                pltpu.make_async_copy(table_h.at[irs[b]], brs[b],
                                      srs[b]).wait()
                pltpu.make_async_copy(table_h.at[ics[b]], bcs[b],
                                      scs[b]).wait()
                pltpu.async_copy(brs[b], outr_h.at[pl.ds(base, CHG)], wrs[b])
                pltpu.async_copy(bcs[b], outc_h.at[pl.ds(base, CHG)], wcs[b])
                pltpu.async_copy(axs[b], aux_h.at[pl.ds(base, CHG)], was[b])

                @pl.when(kk + 2 < n)
                def _():
                    nbase = (start + kk + 2) * CHG
                    pltpu.sync_copy(rowi_h.at[pl.ds(nbase, CHG)], irs[b])
                    pltpu.sync_copy(coli_h.at[pl.ds(nbase, CHG)], ics[b])
                    pltpu.sync_copy(el_h.at[pl.ds(nbase, CHG)], elbs[b])
                    pltpu.sync_copy(et_h.at[pl.ds(nbase, CHG)], etbs[b])
                    wait_rc(b, base)
                    pltpu.async_copy(table_h.at[irs[b]], brs[b], srs[b])
                    pltpu.async_copy(table_h.at[ics[b]], bcs[b], scs[b])

        def body(i2, carry):
            slot(i2, 0)
            slot(i2, 1)
            return carry

        lax.fori_loop(0, trips, body, 0)
        for b in range(2):
            @pl.when(n > b)
            def _():
                last = n - 1 - lax.rem(n - 1 - b, 2)
                lbase = (start + last) * CHG
                wait_rc(b, lbase)
                wait_aux(b, lbase)

    return k(table, x4flat, elflat, eti, batchi, rowi, coli)


def _sc_scatter(m, rc128, rowi, zrows, ne):
    @functools.partial(
        pl.kernel,
        out_type=jax.ShapeDtypeStruct((NC, N, H), jnp.float32),
        mesh=_mesh(),
        scratch_types=[
            pltpu.VMEM_SHARED((N, H), jnp.float32),
            pltpu.VMEM((CH, H), jnp.float32),
            pltpu.VMEM((CH, H), jnp.float32),
            pltpu.VMEM((CH,), jnp.int32),
            pltpu.VMEM((CH,), jnp.int32),
            pltpu.SemaphoreType.DMA,
            pltpu.SemaphoreType.DMA,
        ],
    )
    def k(m_h, rc_h, rowi_h, z_h, out_h, acc, buf0, buf1, iv0, iv1, sa0, sa1):
        bufs = (buf0, buf1)
        ivs = (iv0, iv1)
        sas = (sa0, sa1)
        c = lax.axis_index("c")
        s = lax.axis_index("s")
        pltpu.sync_copy(z_h, acc.at[pl.ds(s * ZB, ZB)])

        @pl.when(s == 0)
        def _():
            pltpu.sync_copy(z_h.at[pl.ds(0, ZREM)], acc.at[pl.ds(NS * ZB, ZREM)])

        plsc.subcore_barrier()
        start, n, trips = _worker_range(s, NS, ne // CH)

        def wait_add(b):
            pltpu.make_async_copy(bufs[b], acc.at[ivs[b]], sas[b]).wait()

        def slot(i2, b, fill):
            kk = i2 * 2 + b

            @pl.when(kk < n)
            def _():
                @pl.when(kk >= 2)
                def _():
                    wait_add(b)

                base = (start + kk) * CH
                pltpu.sync_copy(rowi_h.at[pl.ds(base, CH)], ivs[b])
                fill(base, b)
                pltpu.async_copy(bufs[b], acc.at[ivs[b]], sas[b], add=True)

        def run(fill):
            def body(i2, carry):
                slot(i2, 0, fill)
                slot(i2, 1, fill)
                return carry

            lax.fori_loop(0, trips, body, 0)
            for b in range(2):
                @pl.when(n > b)
                def _():
                    wait_add(b)

        @pl.when(c == 0)
        def _():
            def fill_m(base, b):
                pltpu.sync_copy(m_h.at[pl.ds(base, CH)], bufs[b])

            run(fill_m)

        @pl.when(c == 1)
        def _():
            def fill_rc(base, b):
                pltpu.sync_copy(rc_h.at[pl.ds(base, CH)], bufs[b])

            run(fill_rc)

        plsc.subcore_barrier()
        pltpu.sync_copy(acc.at[pl.ds(s * ZB, ZB)],
                        out_h.at[c, pl.ds(s * ZB, ZB)])

        @pl.when(s == 0)
        def _():
            pltpu.sync_copy(acc.at[pl.ds(NS * ZB, ZREM)],
                            out_h.at[c, pl.ds(NS * ZB, ZREM)])

    return k(m, rc128, rowi, zrows)


def _tc_prelude(time_step2, p):
    def body(ts_r, wt0_r, bt0_r, wt1_r, bt1_r, wtp_r, btp_r, t_out):
        half = H // 2
        i64f = lax.broadcasted_iota(jnp.int32, (1, half), 1).astype(jnp.float32)
        freqs = jnp.exp(-jnp.log(10000.0) * i64f / (half - 1))
        args = ts_r[...].astype(jnp.float32) * freqs
        temb = jnp.concatenate([jnp.sin(args), jnp.cos(args)], axis=1)
        t = jax.nn.relu(_mm(temb, wt0_r[...]) + bt0_r[...])
        t = jax.nn.relu(_mm(t, wt1_r[...]) + bt1_r[...])
        t_out[...] = _mm(t, wtp_r[...]) + btp_r[...]

    return pl.pallas_call(
        body,
        out_shape=jax.ShapeDtypeStruct((G, H), jnp.float32),
    )(time_step2, p["Wt0"], p["bt0"].reshape(1, -1), p["Wt1"],
      p["bt1"].reshape(1, -1), p["Wtp"], p["btp"].reshape(1, -1))


def _tc_node_prep(atom_type, batch2, t, p):
    def body(at_r, b_r, t_r, wa_r, wb_r, bin_r, out_r):
        gi = lax.broadcasted_iota(jnp.int32, (1, G), 1)
        oh = (b_r[...] == gi).astype(jnp.float32)
        tn = _mmsel(oh, t_r[...])
        out_r[...] = _mm(at_r[...], wa_r[...]) + _mm(tn, wb_r[...]) + bin_r[...]

    grid = (N // NBLK,)
    return pl.pallas_call(
        body,
        grid=grid,
        in_specs=[
            pl.BlockSpec((NBLK, H), lambda i: (i, 0)),
            pl.BlockSpec((NBLK, 1), lambda i: (i, 0)),
            pl.BlockSpec((G, H), lambda i: (0, 0)),
            pl.BlockSpec((H, H), lambda i: (0, 0)),
            pl.BlockSpec((H, H), lambda i: (0, 0)),
            pl.BlockSpec((1, H), lambda i: (0, 0)),
        ],
        out_specs=pl.BlockSpec((NBLK, H), lambda i: (i, 0)),
        out_shape=jax.ShapeDtypeStruct((N, H), jnp.float32),
    )(atom_type, batch2, t, p["Win"][:H], p["Win"][H:],
      p["bin"].reshape(1, -1))


def _ea_from_aux(aux, t_r, we1_r, be1_r, we2_r, be2_r, emb_r):
    el = aux[:, 5:6]
    etv = aux[:, 6:7]
    e2g = aux[:, 4:5]
    r1 = jax.nn.relu(el * we1_r[...] + be1_r[...])
    ea0 = _mm(r1, we2_r[...]) + be2_r[...]
    ei = lax.broadcasted_iota(jnp.int32, (1, 8), 1).astype(jnp.float32)
    ohe = (etv == ei).astype(jnp.float32)
    gi = lax.broadcasted_iota(jnp.int32, (1, G), 1).astype(jnp.float32)
    ohg = (e2g == gi).astype(jnp.float32)
    return ea0 * _mmsel(ohe, emb_r[...]) + _mmsel(ohg, t_r[...])


_EAW = None


def _ea_specs():
    W = pl.BlockSpec((H, H), lambda i: (0, 0))
    b1 = pl.BlockSpec((1, H), lambda i: (0, 0))
    return [
        pl.BlockSpec((G, H), lambda i: (0, 0)),
        pl.BlockSpec((1, H), lambda i: (0, 0)),
        b1, W, b1,
        pl.BlockSpec((8, H), lambda i: (0, 0)),
    ]


def _ea_args(t, p):
    return (t, p["We1"], p["be1"].reshape(1, -1), p["We2"],
            p["be2"].reshape(1, -1), p["emb_et"])


def _tc_edge_mlp(hr, hc, aux, t, p, lp):
    def body(hr_r, hc_r, aux_r, t_r, we1_r, be1_r, we2_r, be2_r, emb_r,
             w1r_r, w1c_r, w1d_r, w1e_r, bm1_r, wm2_r, bm2_r,
             wg_r, bg_r, wc1_r, wc2_r, m_out, rc_out):
        aux = aux_r[...]
        rel = aux[:, 0:3]
        d2 = aux[:, 3:4]
        ea = _ea_from_aux(aux, t_r, we1_r, be1_r, we2_r, be2_r, emb_r)
        d2b = d2.astype(jnp.bfloat16).astype(jnp.float32)
        w1db = w1d_r[...].astype(jnp.bfloat16).astype(jnp.float32)
        m = (_mm(hr_r[...], w1r_r[...]) + _mm(hc_r[...], w1c_r[...])
             + d2b * w1db + _mm(ea, w1e_r[...]) + bm1_r[...])
        m = _silu(m)
        m = _silu(_mm(m, wm2_r[...]) + bm2_r[...])
        m = m * jax.nn.sigmoid(_mm(m, wg_r[...]) + bg_r[...])
        coef = _mm(_silu(_mm(m, wc1_r[...])), wc2_r[...])
        reln = rel / (jnp.sqrt(d2) + 1.0)
        m_out[...] = m
        rc_out[...] = jnp.concatenate(
            [reln * coef, jnp.zeros((EBLK, H - 3), jnp.float32)], axis=1)

    ne = hr.shape[0]
    W = pl.BlockSpec((H, H), lambda i: (0, 0))
    b1 = pl.BlockSpec((1, H), lambda i: (0, 0))
    grid = (ne // EBLK,)
    return pl.pallas_call(
        body,
        grid=grid,
        in_specs=[
            pl.BlockSpec((EBLK, H), lambda i: (i, 0)),
            pl.BlockSpec((EBLK, H), lambda i: (i, 0)),
            pl.BlockSpec((EBLK, H), lambda i: (i, 0)),
        ] + _ea_specs() + [
            W, W, b1, W, b1, W, b1,
            pl.BlockSpec((H, 1), lambda i: (0, 0)),
            pl.BlockSpec((1, 1), lambda i: (0, 0)),
            W,
            pl.BlockSpec((H, 1), lambda i: (0, 0)),
        ],
        out_specs=(
            pl.BlockSpec((EBLK, H), lambda i: (i, 0)),
            pl.BlockSpec((EBLK, H), lambda i: (i, 0)),
        ),
        out_shape=(
            jax.ShapeDtypeStruct((ne, H), jnp.float32),
            jax.ShapeDtypeStruct((ne, H), jnp.float32),
        ),
    )(hr, hc, aux, *_ea_args(t, p),
      lp["Wm1"][:H], lp["Wm1"][H:2 * H], lp["Wm1"][2 * H:2 * H + 1],
      lp["Wm1"][2 * H + 1:], lp["bm1"].reshape(1, -1),
      lp["Wm2"], lp["bm2"].reshape(1, -1),
      lp["Wg"], lp["bg"].reshape(1, 1),
      lp["Wc1"], lp["Wc2"])


def _tc_node_update(table, x4, pA, pB, lp):
    def body(tbl_r, x4_r, pa_r, pb_r, whh_r, wha_r, bh1_r, wh2_r, bh2_r,
             h_out, x_out):
        pa = pa_r[...]
        pb = pb_r[...]
        agg = pa[0] + pb[0]
        h = tbl_r[...]
        upd = _mm(_silu(_mm(h, whh_r[...]) + _mm(agg, wha_r[...]) + bh1_r[...]),
                  wh2_r[...]) + bh2_r[...]
        h_out[...] = h + upd
        x_out[...] = x4_r[...] + (pa[1] + pb[1])[:, :4]

    W = pl.BlockSpec((H, H), lambda i: (0, 0))
    b1 = pl.BlockSpec((1, H), lambda i: (0, 0))
    grid = (N // NBLK,)
    return pl.pallas_call(
        body,
        grid=grid,
        in_specs=[
            pl.BlockSpec((NBLK, H), lambda i: (i, 0)),
            pl.BlockSpec((NBLK, 4), lambda i: (i, 0)),
            pl.BlockSpec((NC, NBLK, H), lambda i: (0, i, 0)),
            pl.BlockSpec((NC, NBLK, H), lambda i: (0, i, 0)),
            W, W, b1, W, b1,
        ],
        out_specs=(
            pl.BlockSpec((NBLK, H), lambda i: (i, 0)),
            pl.BlockSpec((NBLK, 4), lambda i: (i, 0)),
        ),
        out_shape=(
            jax.ShapeDtypeStruct((N, H), jnp.float32),
            jax.ShapeDtypeStruct((N, 4), jnp.float32),
        ),
    )(table, x4, pA, pB, lp["Wh1"][:H], lp["Wh1"][H:],
      lp["bh1"].reshape(1, -1), lp["Wh2"], lp["bh2"].reshape(1, -1))


def _tc_final(hr, hc, aux, t, p):
    def body(hr_r, hc_r, aux_r, t_r, we1_r, be1_r, we2_r, be2_r, emb_r,
             wd1h_r, wd1e_r, bd1_r, wd2_r, bd2_r, wd3t_r, bd3_r, out_r):
        ea = _ea_from_aux(aux_r[...], t_r, we1_r, be1_r, we2_r, be2_r, emb_r)
        g1 = jax.nn.relu(_mm(hr_r[...] * hc_r[...], wd1h_r[...])
                         + _mm(ea, wd1e_r[...]) + bd1_r[...])
        g2 = jax.nn.relu(_mm(g1, wd2_r[...]) + bd2_r[...])
        out_r[0] = _mm(wd3t_r[...], g2.T) + bd3_r[...]

    ne = hr.shape[0]
    grid = (ne // EBLK,)
    return pl.pallas_call(
        body,
        grid=grid,
        in_specs=[
            pl.BlockSpec((EBLK, H), lambda i: (i, 0)),
            pl.BlockSpec((EBLK, H), lambda i: (i, 0)),
            pl.BlockSpec((EBLK, H), lambda i: (i, 0)),
        ] + _ea_specs() + [
            pl.BlockSpec((H, H), lambda i: (0, 0)),
            pl.BlockSpec((H, H), lambda i: (0, 0)),
            pl.BlockSpec((1, H), lambda i: (0, 0)),
            pl.BlockSpec((H, H // 2), lambda i: (0, 0)),
            pl.BlockSpec((1, H // 2), lambda i: (0, 0)),
            pl.BlockSpec((1, H // 2), lambda i: (0, 0)),
            pl.BlockSpec((1, 1), lambda i: (0, 0)),
        ],
        out_specs=pl.BlockSpec((1, 1, EBLK), lambda i: (i, 0, 0)),
        out_shape=jax.ShapeDtypeStruct((ne // EBLK, 1, EBLK), jnp.float32),
    )(hr, hc, aux, *_ea_args(t, p),
      p["Wd1"][:H], p["Wd1"][H:], p["bd1"].reshape(1, -1),
      p["Wd2"], p["bd2"].reshape(1, -1), p["Wd3"].reshape(1, -1),
      p["bd3"].reshape(1, 1))


def kernel(atom_type, pos, bond_index, bond_type, batch, time_step, edge_type,
           edge_index, edge_length, params):
    p = params
    row = edge_index[0].astype(jnp.int32)
    col = edge_index[1].astype(jnp.int32)
    batchi = batch.astype(jnp.int32)
    batch2 = batchi.reshape(N, 1)
    ts2 = time_step.astype(jnp.int32).reshape(G, 1)
    eti = edge_type.astype(jnp.int32)
    elflat = edge_length.reshape(E)
    x4 = jnp.pad(pos, ((0, 0), (0, 1)))
    zrows = jnp.zeros((ZB, H), jnp.float32)

    t = _tc_prelude(ts2, p)
    table = _tc_node_prep(atom_type, batch2, t, p)

    E2 = E // 2
    halves = []
    for h0 in (0, E2):
        halves.append(dict(
            row=row[h0:h0 + E2], col=col[h0:h0 + E2],
            el=elflat[h0:h0 + E2], et=eti[h0:h0 + E2]))

    def gath(hv):
        return _sc_gather(table, x4.reshape(-1), hv["el"], hv["et"], batchi,
                          hv["row"], hv["col"], E2)

    for lp in p["layers"]:
        gA = gath(halves[0])
        gB = gath(halves[1])
        mA, rcA = _tc_edge_mlp(*gA, t, p, lp)
        sA = _sc_scatter(mA, rcA, halves[0]["row"], zrows, E2)
        mB, rcB = _tc_edge_mlp(*gB, t, p, lp)
        sB = _sc_scatter(mB, rcB, halves[1]["row"], zrows, E2)
        table, x4 = _tc_node_update(table, x4, sA, sB, lp)

    gA = gath(halves[0])
    gB = gath(halves[1])
    oA = _tc_final(*gA, t, p)
    oB = _tc_final(*gB, t, p)
    edge_inv = jnp.concatenate([oA, oB], axis=0).reshape(E, 1)

    local_edge_mask = edge_type == 0
    return (edge_inv, edge_index, edge_type, edge_length, local_edge_mask)
